# Initial kernel scaffold; baseline (speedup 1.0000x reference)
import jax
import jax.numpy as jnp
from jax.experimental import pallas as pl


def kernel(mesh_features, obj_features, mesh_kinematic, obj_kinematic, index_mm, index_mo, index_om, index_ff, e_mm, e_mo, e_ff, params):
    raise NotImplementedError("write your pallas kernel here")


# speedup vs baseline: 1.2765x; 1.2765x over previous
import functools

import jax
import jax.numpy as jnp
from jax import lax
from jax.experimental import pallas as pl
from jax.experimental.pallas import tpu as pltpu
from jax.experimental.pallas import tpu_sc as plsc

F32 = jnp.float32
LAT = 128
N_MESH, N_OBJ = 10000, 2000
NMP, NOP = 10240, 2048
E_MM_P, E_MO_P, E_FF_P = 163840, 16384, 8192
_NC, _NS, _NW = 2, 16, 32


def _dot(a, b):
    return lax.dot_general(a, b, (((1,), (0,)), ((), ())),
                           preferred_element_type=F32,
                           precision=lax.Precision.DEFAULT)


def _mlp(groups, w1s, b1, w2, b2, w3, b3, ln=None, res=None, block=512):
    xs = [x for g in groups for x in g]
    R = xs[0].shape[0]
    sizes = [len(g) for g in groups]
    has_w = [w is not None for w in w1s]
    ws = [w for w in w1s if w is not None]
    nx, nw = len(xs), len(ws)

    def body(*refs):
        xr = refs[:nx]
        wr = refs[nx:nx + nw]
        b1r, w2r, b2r, w3r, b3r = refs[nx + nw:nx + nw + 5]
        p = nx + nw + 5
        if ln is not None:
            lgr, lbr = refs[p], refs[p + 1]
            p += 2
        if res is not None:
            rr = refs[p]
            p += 1
        out = refs[-1]
        h = None
        k = wi = 0
        for gi, sz in enumerate(sizes):
            acc = xr[k][...]
            for j in range(1, sz):
                acc = acc + xr[k + j][...]
            k += sz
            if has_w[gi]:
                acc = _dot(acc, wr[wi][...])
                wi += 1
            h = acc if h is None else h + acc
        h = jnp.maximum(h + b1r[...], 0.0)
        h = jnp.maximum(_dot(h, w2r[...]) + b2r[...], 0.0)
        y = _dot(h, w3r[...]) + b3r[...]
        if ln is not None:
            mu = jnp.mean(y, axis=-1, keepdims=True)
            var = jnp.mean((y - mu) ** 2, axis=-1, keepdims=True)
            y = (y - mu) * lax.rsqrt(var + 1e-5) * lgr[...] + lbr[...]
        if res is not None:
            y = y + rr[...]
        out[...] = y

    args = list(xs) + list(ws) + [b1.reshape(1, -1), w2, b2.reshape(1, -1),
                                  w3, b3.reshape(1, -1)]
    if ln is not None:
        args += [ln[0].reshape(1, -1), ln[1].reshape(1, -1)]
    if res is not None:
        args.append(res)
    in_specs = [pl.BlockSpec((block, x.shape[1]), lambda i: (i, 0)) for x in xs]
    in_specs += [pl.BlockSpec(w.shape, lambda i: (0, 0)) for w in ws]
    in_specs += [pl.BlockSpec(a.shape, lambda i: (0, 0)) for a in args[nx + nw:nx + nw + 5]]
    if ln is not None:
        in_specs += [pl.BlockSpec((1, LAT), lambda i: (0, 0))] * 2
    if res is not None:
        in_specs.append(pl.BlockSpec((block, LAT), lambda i: (i, 0)))
    return pl.pallas_call(
        body,
        grid=(R // block,),
        in_specs=in_specs,
        out_specs=pl.BlockSpec((block, LAT), lambda i: (i, 0)),
        out_shape=jax.ShapeDtypeStruct((R, LAT), F32),
    )(*args)


def _tables(x, ws, block=512):
    R = x.shape[0]
    nw = len(ws)

    def body(*refs):
        xv = refs[0][...]
        for i in range(nw):
            refs[1 + nw + i][...] = _dot(xv, refs[1 + i][...])

    return pl.pallas_call(
        body,
        grid=(R // block,),
        in_specs=[pl.BlockSpec((block, LAT), lambda i: (i, 0))]
        + [pl.BlockSpec((LAT, LAT), lambda i: (0, 0))] * nw,
        out_specs=[pl.BlockSpec((block, LAT), lambda i: (i, 0))] * nw,
        out_shape=[jax.ShapeDtypeStruct((R, LAT), F32)] * nw,
    )(x, *ws)


def _kch(rows):
    return 4 if rows % 4 == 0 else (2 if rows % 2 == 0 else 1)


def _sc_gather_all(tables, idxs2d):
    nj = len(tables)
    epads = [i.shape[0] * 128 for i in idxs2d]
    mesh = plsc.VectorSubcoreMesh(core_axis_name="c", subcore_axis_name="s")

    @functools.partial(
        pl.kernel, mesh=mesh,
        out_type=[jax.ShapeDtypeStruct((e, LAT), F32) for e in epads],
        scratch_types=[
            pltpu.VMEM((4, 128), jnp.int32),
            pltpu.VMEM((512, LAT), F32),
            pltpu.SemaphoreType.DMA,
        ],
    )
    def k(*refs):
        trefs = refs[:nj]
        irefs = refs[nj:2 * nj]
        orefs = refs[2 * nj:3 * nj]
        idx_v, rows_v, sem = refs[3 * nj:]
        wid = lax.axis_index("s") * _NC + lax.axis_index("c")
        for j in range(nj):
            rows = epads[j] // 128 // _NW
            base = wid * rows
            kch = _kch(rows)

            def gbody(g, _, j=j, base=base, kch=kch):
                r0 = base + g * kch
                pltpu.sync_copy(irefs[j].at[pl.ds(r0, kch)],
                                idx_v.at[pl.ds(0, kch)])
                cps = [pltpu.async_copy(trefs[j].at[idx_v.at[q]],
                                        rows_v.at[pl.ds(q * 128, 128)], sem)
                       for q in range(kch)]
                for cp in cps:
                    cp.wait()
                pltpu.sync_copy(rows_v.at[pl.ds(0, kch * 128)],
                                orefs[j].at[pl.ds(r0 * 128, kch * 128)])
                return 0

            lax.fori_loop(0, rows // kch, gbody, 0)

    return k(*tables, *idxs2d)


def _sc_scatter_all(zeros, vals, idxs2d, out_rows):
    nj = len(vals)
    epads = [i.shape[0] * 128 for i in idxs2d]
    mesh = plsc.VectorSubcoreMesh(core_axis_name="c", subcore_axis_name="s")

    @functools.partial(
        pl.kernel, mesh=mesh,
        out_type=[jax.ShapeDtypeStruct((2, r, LAT), F32) for r in out_rows],
        scratch_types=[
            pltpu.VMEM_SHARED((NMP, LAT), F32),
            pltpu.VMEM((4, 128), jnp.int32),
            pltpu.VMEM((256, LAT), F32),
            pltpu.SemaphoreType.DMA,
        ],
    )
    def k(*refs):
        zr = refs[0]
        vrefs = refs[1:1 + nj]
        irefs = refs[1 + nj:1 + 2 * nj]
        orefs = refs[1 + 2 * nj:1 + 3 * nj]
        acc, idx_v, buf, sem = refs[1 + 3 * nj:]
        c = lax.axis_index("c")
        s = lax.axis_index("s")
        wid = s * _NC + c
        for j in range(nj):
            rpt = out_rows[j] // _NS
            pltpu.sync_copy(zr, buf.at[pl.ds(0, 128)])
            for r in range(rpt // 128):
                pltpu.sync_copy(buf.at[pl.ds(0, 128)],
                                acc.at[pl.ds(s * rpt + r * 128, 128)])
            plsc.subcore_barrier()
            rows = epads[j] // 128 // _NW
            base = wid * rows
            kch = min(_kch(rows), 2)

            def sbody(g, _, j=j, base=base, kch=kch):
                r0 = base + g * kch
                pltpu.sync_copy(irefs[j].at[pl.ds(r0, kch)],
                                idx_v.at[pl.ds(0, kch)])
                pltpu.sync_copy(vrefs[j].at[pl.ds(r0 * 128, kch * 128)],
                                buf.at[pl.ds(0, kch * 128)])
                for q in range(kch):
                    pltpu.sync_copy(buf.at[pl.ds(q * 128, 128)],
                                    acc.at[idx_v.at[q]], add=True)
                return 0

            lax.fori_loop(0, rows // kch, sbody, 0)
            plsc.subcore_barrier()
            for r in range(rpt // 128):
                r0 = s * rpt + r * 128
                pltpu.sync_copy(acc.at[pl.ds(r0, 128)],
                                buf.at[pl.ds(0, 128)])
                pltpu.sync_copy(buf.at[pl.ds(0, 128)],
                                orefs[j].at[c, pl.ds(r0, 128)])
            plsc.subcore_barrier()

    return k(zeros, *vals, *idxs2d)


def _pad_rows(x, rows, fill=0.0):
    return jnp.pad(x, ((0, rows - x.shape[0]), (0, 0)), constant_values=fill)


def _pad_idx(idx, n, fill):
    return jnp.pad(idx, (0, n - idx.shape[0]), constant_values=fill)


def _mlp_params(p):
    (w1, b1), (w2, b2), (w3, b3) = p["layers"]
    return w1, b1, w2, b2, w3, b3


def kernel(mesh_features, obj_features, mesh_kinematic, obj_kinematic,
           index_mm, index_mo, index_om, index_ff, e_mm, e_mo, e_ff, params):
    pm = params
    m_kin = jax.nn.one_hot(mesh_kinematic, 3, dtype=F32)
    o_kin = jax.nn.one_hot(obj_kinematic, 3, dtype=F32)
    m_in = jnp.concatenate([mesh_features, m_kin], axis=-1)
    o_in = jnp.concatenate([obj_features, o_kin], axis=-1)
    m_in = (m_in - pm["node_mean"]) / pm["node_std"]
    o_in = (o_in - pm["node_mean"]) / pm["node_std"]
    nk = jax.random.key(42)
    m_in = m_in.at[:, :3].add(
        1e-05 * jax.random.normal(jax.random.fold_in(nk, 0), (N_MESH, 3), F32))
    o_in = o_in.at[:, :3].add(
        1e-05 * jax.random.normal(jax.random.fold_in(nk, 1), (N_OBJ, 3), F32))
    m_in = _pad_rows(jnp.pad(m_in, ((0, 0), (0, 5))), NMP)
    o_in = _pad_rows(jnp.pad(o_in, ((0, 0), (0, 5))), NOP)

    e_mm_p = _pad_rows(e_mm, E_MM_P)
    e_mo_p = _pad_rows(e_mo, E_MO_P)
    e_ff_p = _pad_rows(jnp.pad(e_ff, ((0, 0), (0, 6))), E_FF_P)

    imm0 = _pad_idx(index_mm[0], E_MM_P, 0)
    imm1 = _pad_idx(index_mm[1], E_MM_P, N_MESH)
    imo0 = _pad_idx(index_mo[0], E_MO_P, 0)
    imo1 = _pad_idx(index_mo[1], E_MO_P, N_OBJ)
    iom0 = _pad_idx(index_om[0], E_MO_P, 0)
    iom1 = _pad_idx(index_om[1], E_MO_P, N_MESH)
    iff0 = _pad_idx(index_ff[0], E_FF_P, 0)
    iff1 = _pad_idx(index_ff[1], E_FF_P, N_MESH)
    i2 = lambda ix: ix.reshape(-1, 128)
    gidx = [i2(imm0), i2(imm1), i2(imo0), i2(imo1),
            i2(iom0), i2(iom1), i2(iff0), i2(iff1)]
    sidx = [i2(imm1), i2(iom1), i2(iff1), i2(imo1)]
    zeros128 = jnp.zeros((128, LAT), F32)

    def fold(enc, mean, std, pad_to):
        w1, b1, w2, b2, w3, b3 = _mlp_params(enc)
        w1f = w1 / std[:, None]
        b1f = b1 - (mean / std) @ w1
        w1f = jnp.pad(w1f, ((0, pad_to - w1f.shape[0]), (0, 0)))
        return w1f, b1f, w2, b2, w3, b3

    w1, b1, w2, b2, w3, b3 = _mlp_params(pm["enc_mesh"])
    m = _mlp([[m_in]], [jnp.pad(w1, ((0, 5), (0, 0)))], b1, w2, b2, w3, b3,
             ln=pm["enc_mesh"]["ln"])
    w1, b1, w2, b2, w3, b3 = _mlp_params(pm["enc_obj"])
    o = _mlp([[o_in]], [jnp.pad(w1, ((0, 5), (0, 0)))], b1, w2, b2, w3, b3,
             ln=pm["enc_obj"]["ln"])
    w1f, b1f, w2, b2, w3, b3 = fold(pm["enc_mm"], pm["edge_mean"], pm["edge_std"], 8)
    lmm = _mlp([[e_mm_p]], [w1f], b1f, w2, b2, w3, b3, ln=pm["enc_mm"]["ln"])
    w1f, b1f, w2, b2, w3, b3 = fold(pm["enc_mo"], pm["edge_mean"], pm["edge_std"], 8)
    lmo = _mlp([[e_mo_p]], [w1f], b1f, w2, b2, w3, b3, ln=pm["enc_mo"]["ln"])
    w1f, b1f, w2, b2, w3, b3 = fold(pm["enc_om"], pm["edge_mean"], pm["edge_std"], 8)
    lom = _mlp([[e_mo_p]], [w1f], b1f, w2, b2, w3, b3, ln=pm["enc_om"]["ln"])
    w1f, b1f, w2, b2, w3, b3 = fold(pm["enc_ff"], pm["face_mean"], pm["face_std"], 40)
    lff = _mlp([[e_ff_p]], [w1f], b1f, w2, b2, w3, b3, ln=pm["enc_ff"]["ln"])

    for sp in pm["steps"]:
        wmm = _mlp_params(sp["mm"])
        wmo = _mlp_params(sp["mo"])
        wom = _mlp_params(sp["om"])
        wff = _mlp_params(sp["ff"])
        tm = _tables(m, [wmm[0][:128], wmm[0][128:256], wmo[0][:128],
                         wom[0][128:256], wff[0][:128], wff[0][128:256]])
        to = _tables(o, [wmo[0][128:256], wom[0][:128]])
        g = _sc_gather_all(
            [tm[0], tm[1], tm[2], to[0], to[1], tm[3], tm[4], tm[5]], gidx)
        lmm = _mlp([[g[0], g[1]], [lmm]], [None, wmm[0][256:]], wmm[1],
                   wmm[2], wmm[3], wmm[4], wmm[5],
                   ln=sp["mm"]["ln"], res=lmm)
        lmo = _mlp([[g[2], g[3]], [lmo]], [None, wmo[0][256:]], wmo[1],
                   wmo[2], wmo[3], wmo[4], wmo[5],
                   ln=sp["mo"]["ln"], res=lmo)
        lom = _mlp([[g[4], g[5]], [lom]], [None, wom[0][256:]], wom[1],
                   wom[2], wom[3], wom[4], wom[5],
                   ln=sp["om"]["ln"], res=lom)
        lff = _mlp([[g[6], g[7]], [lff]], [None, wff[0][256:]], wff[1],
                   wff[2], wff[3], wff[4], wff[5],
                   ln=sp["ff"]["ln"], res=lff)
        pmm, pom, pff, pmo = _sc_scatter_all(
            zeros128, [lmm, lom, lff, lmo], sidx, [NMP, NMP, NMP, NOP])
        wn = _mlp_params(sp["mesh_node"])
        m = _mlp([[m], [pmm[0], pmm[1]], [pom[0], pom[1]], [pff[0], pff[1]]],
                 [wn[0][:128], wn[0][128:256], wn[0][256:384], wn[0][384:]],
                 wn[1], wn[2], wn[3], wn[4], wn[5],
                 ln=sp["mesh_node"]["ln"], res=m)
        wo = _mlp_params(sp["obj_node"])
        o = _mlp([[o], [pmo[0], pmo[1]]], [wo[0][:128], wo[0][128:]],
                 wo[1], wo[2], wo[3], wo[4], wo[5],
                 ln=sp["obj_node"]["ln"], res=o)

    w1, b1, w2, b2, w3, b3 = _mlp_params(pm["dec_mesh"])
    md = _mlp([[m]], [w1], b1, w2, b2,
              jnp.pad(w3, ((0, 0), (0, 125))), jnp.pad(b3, (0, 125)))
    w1, b1, w2, b2, w3, b3 = _mlp_params(pm["dec_obj"])
    od = _mlp([[o]], [w1], b1, w2, b2,
              jnp.pad(w3, ((0, 0), (0, 125))), jnp.pad(b3, (0, 125)))
    return md[:N_MESH, :3], od[:N_OBJ, :3]


# speedup vs baseline: 1.3336x; 1.0447x over previous
import functools

import jax
import jax.numpy as jnp
from jax import lax
from jax.experimental import pallas as pl
from jax.experimental.pallas import tpu as pltpu
from jax.experimental.pallas import tpu_sc as plsc

F32 = jnp.float32
LAT = 128
N_MESH, N_OBJ = 10000, 2000
NMP, NOP = 10240, 2048
E_MM_P, E_MO_P, E_FF_P = 163840, 16384, 8192
_NC, _NS, _NW = 2, 16, 32


def _dot(a, b):
    return lax.dot_general(a, b, (((1,), (0,)), ((), ())),
                           preferred_element_type=F32,
                           precision=lax.Precision.DEFAULT)


def _mlp(groups, w1s, b1, w2, b2, w3, b3, ln=None, res=None, block=512):
    xs = [x for g in groups for x in g]
    R = xs[0].shape[0]
    sizes = [len(g) for g in groups]
    has_w = [w is not None for w in w1s]
    ws = [w for w in w1s if w is not None]
    nx, nw = len(xs), len(ws)

    def body(*refs):
        xr = refs[:nx]
        wr = refs[nx:nx + nw]
        b1r, w2r, b2r, w3r, b3r = refs[nx + nw:nx + nw + 5]
        p = nx + nw + 5
        if ln is not None:
            lgr, lbr = refs[p], refs[p + 1]
            p += 2
        if res is not None:
            rr = refs[p]
            p += 1
        out = refs[-1]
        h = None
        k = wi = 0
        for gi, sz in enumerate(sizes):
            acc = xr[k][...]
            for j in range(1, sz):
                acc = acc + xr[k + j][...]
            k += sz
            if has_w[gi]:
                acc = _dot(acc, wr[wi][...])
                wi += 1
            h = acc if h is None else h + acc
        h = jnp.maximum(h + b1r[...], 0.0)
        h = jnp.maximum(_dot(h, w2r[...]) + b2r[...], 0.0)
        y = _dot(h, w3r[...]) + b3r[...]
        if ln is not None:
            mu = jnp.mean(y, axis=-1, keepdims=True)
            var = jnp.mean((y - mu) ** 2, axis=-1, keepdims=True)
            y = (y - mu) * lax.rsqrt(var + 1e-5) * lgr[...] + lbr[...]
        if res is not None:
            y = y + rr[...]
        out[...] = y

    args = list(xs) + list(ws) + [b1.reshape(1, -1), w2, b2.reshape(1, -1),
                                  w3, b3.reshape(1, -1)]
    if ln is not None:
        args += [ln[0].reshape(1, -1), ln[1].reshape(1, -1)]
    if res is not None:
        args.append(res)
    in_specs = [pl.BlockSpec((block, x.shape[1]), lambda i: (i, 0)) for x in xs]
    in_specs += [pl.BlockSpec(w.shape, lambda i: (0, 0)) for w in ws]
    in_specs += [pl.BlockSpec(a.shape, lambda i: (0, 0)) for a in args[nx + nw:nx + nw + 5]]
    if ln is not None:
        in_specs += [pl.BlockSpec((1, LAT), lambda i: (0, 0))] * 2
    if res is not None:
        in_specs.append(pl.BlockSpec((block, LAT), lambda i: (i, 0)))
    return pl.pallas_call(
        body,
        grid=(R // block,),
        in_specs=in_specs,
        out_specs=pl.BlockSpec((block, LAT), lambda i: (i, 0)),
        out_shape=jax.ShapeDtypeStruct((R, LAT), F32),
    )(*args)


def _tables(x, ws, block=512):
    R = x.shape[0]
    nw = len(ws)

    def body(*refs):
        xv = refs[0][...]
        for i in range(nw):
            refs[1 + nw + i][...] = _dot(xv, refs[1 + i][...])

    return pl.pallas_call(
        body,
        grid=(R // block,),
        in_specs=[pl.BlockSpec((block, LAT), lambda i: (i, 0))]
        + [pl.BlockSpec((LAT, LAT), lambda i: (0, 0))] * nw,
        out_specs=[pl.BlockSpec((block, LAT), lambda i: (i, 0))] * nw,
        out_shape=[jax.ShapeDtypeStruct((R, LAT), F32)] * nw,
    )(x, *ws)


def _sc_gather_all(tables, idxs2d):
    nj = len(tables)
    epads = [i.shape[0] * 128 for i in idxs2d]
    rows_w = [e // 128 // _NW for e in epads]
    joff = [0]
    for r in rows_w:
        joff.append(joff[-1] + r)
    mesh = plsc.VectorSubcoreMesh(core_axis_name="c", subcore_axis_name="s")

    @functools.partial(
        pl.kernel, mesh=mesh,
        out_type=[jax.ShapeDtypeStruct((e, LAT), F32) for e in epads],
        scratch_types=[
            pltpu.VMEM((joff[-1], 128), jnp.int32),
            pltpu.VMEM((512, LAT), F32),
            pltpu.SemaphoreType.DMA,
            pltpu.SemaphoreType.DMA((2,)),
        ],
    )
    def k(*refs):
        trefs = refs[:nj]
        irefs = refs[nj:2 * nj]
        orefs = refs[2 * nj:3 * nj]
        idx_all, rows_v, gsem, wsem = refs[3 * nj:]
        wid = lax.axis_index("s") * _NC + lax.axis_index("c")
        icps = [pltpu.async_copy(irefs[j].at[pl.ds(wid * rows_w[j], rows_w[j])],
                                 idx_all.at[pl.ds(joff[j], rows_w[j])], gsem)
                for j in range(nj)]
        for cp in icps:
            cp.wait()
        for j in range(nj):
            rows = rows_w[j]
            base = wid * rows
            nsg = rows // 2

            def gbody(sg, _, j=j, base=base):
                h = sg % 2

                @pl.when(sg >= 2)
                def _():
                    pltpu.make_async_copy(
                        rows_v.at[pl.ds(h * 256, 256)],
                        orefs[j].at[pl.ds((base + (sg - 2) * 2) * 128, 256)],
                        wsem.at[h]).wait()

                cps = [pltpu.async_copy(
                    trefs[j].at[idx_all.at[joff[j] + sg * 2 + q]],
                    rows_v.at[pl.ds(h * 256 + q * 128, 128)], gsem)
                    for q in (0, 1)]
                for cp in cps:
                    cp.wait()
                pltpu.async_copy(rows_v.at[pl.ds(h * 256, 256)],
                                 orefs[j].at[pl.ds((base + sg * 2) * 128, 256)],
                                 wsem.at[h])
                return 0

            lax.fori_loop(0, nsg, gbody, 0)
            for t in range(max(nsg - 2, 0), nsg):
                pltpu.make_async_copy(
                    rows_v.at[pl.ds((t % 2) * 256, 256)],
                    orefs[j].at[pl.ds((base + t * 2) * 128, 256)],
                    wsem.at[t % 2]).wait()

    return k(*tables, *idxs2d)


def _sc_scatter_all(zeros, vals, idxs2d, out_rows):
    nj = len(vals)
    epads = [i.shape[0] * 128 for i in idxs2d]
    mesh = plsc.VectorSubcoreMesh(core_axis_name="c", subcore_axis_name="s")

    rows_w = [e // 128 // _NW for e in epads]
    joff = [0]
    for r in rows_w:
        joff.append(joff[-1] + r)

    @functools.partial(
        pl.kernel, mesh=mesh,
        out_type=[jax.ShapeDtypeStruct((2, r, LAT), F32) for r in out_rows],
        scratch_types=[
            pltpu.VMEM_SHARED((NMP, LAT), F32),
            pltpu.VMEM((joff[-1], 128), jnp.int32),
            pltpu.VMEM((256, LAT), F32),
            pltpu.SemaphoreType.DMA,
            pltpu.SemaphoreType.DMA((2,)),
        ],
    )
    def k(*refs):
        zr = refs[0]
        vrefs = refs[1:1 + nj]
        irefs = refs[1 + nj:1 + 2 * nj]
        orefs = refs[1 + 2 * nj:1 + 3 * nj]
        acc, idx_all, buf, sem, lsem = refs[1 + 3 * nj:]
        c = lax.axis_index("c")
        s = lax.axis_index("s")
        wid = s * _NC + c
        icps = [pltpu.async_copy(irefs[j].at[pl.ds(wid * rows_w[j], rows_w[j])],
                                 idx_all.at[pl.ds(joff[j], rows_w[j])], sem)
                for j in range(nj)]
        for cp in icps:
            cp.wait()
        for j in range(nj):
            rpt = out_rows[j] // _NS
            pltpu.sync_copy(zr, buf.at[pl.ds(0, 128)])
            for r in range(rpt // 128):
                pltpu.sync_copy(buf.at[pl.ds(0, 128)],
                                acc.at[pl.ds(s * rpt + r * 128, 128)])
            plsc.subcore_barrier()
            rows = rows_w[j]
            base = wid * rows
            pltpu.async_copy(vrefs[j].at[pl.ds(base * 128, 128)],
                             buf.at[pl.ds(0, 128)], lsem.at[0])

            def sbody(g, _, j=j, base=base, rows=rows):
                h = g % 2
                pltpu.make_async_copy(
                    vrefs[j].at[pl.ds((base + g) * 128, 128)],
                    buf.at[pl.ds(h * 128, 128)], lsem.at[h]).wait()

                @pl.when(g + 1 < rows)
                def _():
                    pltpu.async_copy(
                        vrefs[j].at[pl.ds((base + g + 1) * 128, 128)],
                        buf.at[pl.ds((1 - h) * 128, 128)], lsem.at[1 - h])

                pltpu.sync_copy(buf.at[pl.ds(h * 128, 128)],
                                acc.at[idx_all.at[joff[j] + g]], add=True)
                return 0

            lax.fori_loop(0, rows, sbody, 0)
            plsc.subcore_barrier()
            for r in range(rpt // 128):
                r0 = s * rpt + r * 128
                pltpu.sync_copy(acc.at[pl.ds(r0, 128)],
                                buf.at[pl.ds(0, 128)])
                pltpu.sync_copy(buf.at[pl.ds(0, 128)],
                                orefs[j].at[c, pl.ds(r0, 128)])
            plsc.subcore_barrier()

    return k(zeros, *vals, *idxs2d)


def _pad_rows(x, rows, fill=0.0):
    return jnp.pad(x, ((0, rows - x.shape[0]), (0, 0)), constant_values=fill)


def _pad_idx(idx, n, fill):
    return jnp.pad(idx, (0, n - idx.shape[0]), constant_values=fill)


def _mlp_params(p):
    (w1, b1), (w2, b2), (w3, b3) = p["layers"]
    return w1, b1, w2, b2, w3, b3


def kernel(mesh_features, obj_features, mesh_kinematic, obj_kinematic,
           index_mm, index_mo, index_om, index_ff, e_mm, e_mo, e_ff, params):
    pm = params
    m_kin = jax.nn.one_hot(mesh_kinematic, 3, dtype=F32)
    o_kin = jax.nn.one_hot(obj_kinematic, 3, dtype=F32)
    m_in = jnp.concatenate([mesh_features, m_kin], axis=-1)
    o_in = jnp.concatenate([obj_features, o_kin], axis=-1)
    m_in = (m_in - pm["node_mean"]) / pm["node_std"]
    o_in = (o_in - pm["node_mean"]) / pm["node_std"]
    nk = jax.random.key(42)
    m_in = m_in.at[:, :3].add(
        1e-05 * jax.random.normal(jax.random.fold_in(nk, 0), (N_MESH, 3), F32))
    o_in = o_in.at[:, :3].add(
        1e-05 * jax.random.normal(jax.random.fold_in(nk, 1), (N_OBJ, 3), F32))
    m_in = _pad_rows(jnp.pad(m_in, ((0, 0), (0, 5))), NMP)
    o_in = _pad_rows(jnp.pad(o_in, ((0, 0), (0, 5))), NOP)

    e_mm_p = _pad_rows(e_mm, E_MM_P)
    e_mo_p = _pad_rows(e_mo, E_MO_P)
    e_ff_p = _pad_rows(jnp.pad(e_ff, ((0, 0), (0, 6))), E_FF_P)

    imm0 = _pad_idx(index_mm[0], E_MM_P, 0)
    imm1 = _pad_idx(index_mm[1], E_MM_P, N_MESH)
    imo0 = _pad_idx(index_mo[0], E_MO_P, 0)
    imo1 = _pad_idx(index_mo[1], E_MO_P, N_OBJ)
    iom0 = _pad_idx(index_om[0], E_MO_P, 0)
    iom1 = _pad_idx(index_om[1], E_MO_P, N_MESH)
    iff0 = _pad_idx(index_ff[0], E_FF_P, 0)
    iff1 = _pad_idx(index_ff[1], E_FF_P, N_MESH)
    i2 = lambda ix: ix.reshape(-1, 128)
    gidx = [i2(imm0), i2(imm1), i2(imo0), i2(imo1),
            i2(iom0), i2(iom1), i2(iff0), i2(iff1)]
    sidx = [i2(imm1), i2(iom1), i2(iff1), i2(imo1)]
    zeros128 = jnp.zeros((128, LAT), F32)

    def fold(enc, mean, std, pad_to):
        w1, b1, w2, b2, w3, b3 = _mlp_params(enc)
        w1f = w1 / std[:, None]
        b1f = b1 - (mean / std) @ w1
        w1f = jnp.pad(w1f, ((0, pad_to - w1f.shape[0]), (0, 0)))
        return w1f, b1f, w2, b2, w3, b3

    w1, b1, w2, b2, w3, b3 = _mlp_params(pm["enc_mesh"])
    m = _mlp([[m_in]], [jnp.pad(w1, ((0, 5), (0, 0)))], b1, w2, b2, w3, b3,
             ln=pm["enc_mesh"]["ln"])
    w1, b1, w2, b2, w3, b3 = _mlp_params(pm["enc_obj"])
    o = _mlp([[o_in]], [jnp.pad(w1, ((0, 5), (0, 0)))], b1, w2, b2, w3, b3,
             ln=pm["enc_obj"]["ln"])
    w1f, b1f, w2, b2, w3, b3 = fold(pm["enc_mm"], pm["edge_mean"], pm["edge_std"], 8)
    lmm = _mlp([[e_mm_p]], [w1f], b1f, w2, b2, w3, b3, ln=pm["enc_mm"]["ln"])
    w1f, b1f, w2, b2, w3, b3 = fold(pm["enc_mo"], pm["edge_mean"], pm["edge_std"], 8)
    lmo = _mlp([[e_mo_p]], [w1f], b1f, w2, b2, w3, b3, ln=pm["enc_mo"]["ln"])
    w1f, b1f, w2, b2, w3, b3 = fold(pm["enc_om"], pm["edge_mean"], pm["edge_std"], 8)
    lom = _mlp([[e_mo_p]], [w1f], b1f, w2, b2, w3, b3, ln=pm["enc_om"]["ln"])
    w1f, b1f, w2, b2, w3, b3 = fold(pm["enc_ff"], pm["face_mean"], pm["face_std"], 40)
    lff = _mlp([[e_ff_p]], [w1f], b1f, w2, b2, w3, b3, ln=pm["enc_ff"]["ln"])

    for sp in pm["steps"]:
        wmm = _mlp_params(sp["mm"])
        wmo = _mlp_params(sp["mo"])
        wom = _mlp_params(sp["om"])
        wff = _mlp_params(sp["ff"])
        tm = _tables(m, [wmm[0][:128], wmm[0][128:256], wmo[0][:128],
                         wom[0][128:256], wff[0][:128], wff[0][128:256]])
        to = _tables(o, [wmo[0][128:256], wom[0][:128]])
        g = _sc_gather_all(
            [tm[0], tm[1], tm[2], to[0], to[1], tm[3], tm[4], tm[5]], gidx)
        lmm = _mlp([[g[0], g[1]], [lmm]], [None, wmm[0][256:]], wmm[1],
                   wmm[2], wmm[3], wmm[4], wmm[5],
                   ln=sp["mm"]["ln"], res=lmm)
        lmo = _mlp([[g[2], g[3]], [lmo]], [None, wmo[0][256:]], wmo[1],
                   wmo[2], wmo[3], wmo[4], wmo[5],
                   ln=sp["mo"]["ln"], res=lmo)
        lom = _mlp([[g[4], g[5]], [lom]], [None, wom[0][256:]], wom[1],
                   wom[2], wom[3], wom[4], wom[5],
                   ln=sp["om"]["ln"], res=lom)
        lff = _mlp([[g[6], g[7]], [lff]], [None, wff[0][256:]], wff[1],
                   wff[2], wff[3], wff[4], wff[5],
                   ln=sp["ff"]["ln"], res=lff)
        pmm, pom, pff, pmo = _sc_scatter_all(
            zeros128, [lmm, lom, lff, lmo], sidx, [NMP, NMP, NMP, NOP])
        wn = _mlp_params(sp["mesh_node"])
        m = _mlp([[m], [pmm[0], pmm[1]], [pom[0], pom[1]], [pff[0], pff[1]]],
                 [wn[0][:128], wn[0][128:256], wn[0][256:384], wn[0][384:]],
                 wn[1], wn[2], wn[3], wn[4], wn[5],
                 ln=sp["mesh_node"]["ln"], res=m)
        wo = _mlp_params(sp["obj_node"])
        o = _mlp([[o], [pmo[0], pmo[1]]], [wo[0][:128], wo[0][128:]],
                 wo[1], wo[2], wo[3], wo[4], wo[5],
                 ln=sp["obj_node"]["ln"], res=o)

    w1, b1, w2, b2, w3, b3 = _mlp_params(pm["dec_mesh"])
    md = _mlp([[m]], [w1], b1, w2, b2,
              jnp.pad(w3, ((0, 0), (0, 125))), jnp.pad(b3, (0, 125)))
    w1, b1, w2, b2, w3, b3 = _mlp_params(pm["dec_obj"])
    od = _mlp([[o]], [w1], b1, w2, b2,
              jnp.pad(w3, ((0, 0), (0, 125))), jnp.pad(b3, (0, 125)))
    return md[:N_MESH, :3], od[:N_OBJ, :3]


# speedup vs baseline: 2.0010x; 1.5005x over previous
import functools

import jax
import jax.numpy as jnp
from jax import lax
from jax.experimental import pallas as pl
from jax.experimental.pallas import tpu as pltpu
from jax.experimental.pallas import tpu_sc as plsc

F32 = jnp.float32
LAT = 128
N_MESH, N_OBJ = 10000, 2000
NMP, NOP = 10240, 2048
E_MM_P, E_MO_P, E_FF_P = 163840, 16384, 8192
_NC, _NS, _NW = 2, 16, 32


def _dot(a, b):
    return lax.dot_general(a, b, (((1,), (0,)), ((), ())),
                           preferred_element_type=F32,
                           precision=lax.Precision.DEFAULT)


def _mlp(groups, w1s, b1, w2, b2, w3, b3, ln=None, res=None, block=512):
    xs = [x for g in groups for x in g]
    R = xs[0].shape[0]
    sizes = [len(g) for g in groups]
    has_w = [w is not None for w in w1s]
    ws = [w for w in w1s if w is not None]
    nx, nw = len(xs), len(ws)

    def body(*refs):
        xr = refs[:nx]
        wr = refs[nx:nx + nw]
        b1r, w2r, b2r, w3r, b3r = refs[nx + nw:nx + nw + 5]
        p = nx + nw + 5
        if ln is not None:
            lgr, lbr = refs[p], refs[p + 1]
            p += 2
        if res is not None:
            rr = refs[p]
            p += 1
        out = refs[-1]
        h = None
        k = wi = 0
        for gi, sz in enumerate(sizes):
            acc = xr[k][...]
            for j in range(1, sz):
                acc = acc + xr[k + j][...]
            k += sz
            if has_w[gi]:
                acc = _dot(acc, wr[wi][...])
                wi += 1
            h = acc if h is None else h + acc
        h = jnp.maximum(h + b1r[...], 0.0)
        h = jnp.maximum(_dot(h, w2r[...]) + b2r[...], 0.0)
        y = _dot(h, w3r[...]) + b3r[...]
        if ln is not None:
            mu = jnp.mean(y, axis=-1, keepdims=True)
            var = jnp.mean((y - mu) ** 2, axis=-1, keepdims=True)
            y = (y - mu) * lax.rsqrt(var + 1e-5) * lgr[...] + lbr[...]
        if res is not None:
            y = y + rr[...]
        out[...] = y

    args = list(xs) + list(ws) + [b1.reshape(1, -1), w2, b2.reshape(1, -1),
                                  w3, b3.reshape(1, -1)]
    if ln is not None:
        args += [ln[0].reshape(1, -1), ln[1].reshape(1, -1)]
    if res is not None:
        args.append(res)
    in_specs = [pl.BlockSpec((block, x.shape[1]), lambda i: (i, 0)) for x in xs]
    in_specs += [pl.BlockSpec(w.shape, lambda i: (0, 0)) for w in ws]
    in_specs += [pl.BlockSpec(a.shape, lambda i: (0, 0)) for a in args[nx + nw:nx + nw + 5]]
    if ln is not None:
        in_specs += [pl.BlockSpec((1, LAT), lambda i: (0, 0))] * 2
    if res is not None:
        in_specs.append(pl.BlockSpec((block, LAT), lambda i: (i, 0)))
    return pl.pallas_call(
        body,
        grid=(R // block,),
        in_specs=in_specs,
        out_specs=pl.BlockSpec((block, LAT), lambda i: (i, 0)),
        out_shape=jax.ShapeDtypeStruct((R, LAT), F32),
    )(*args)


def _tables(x, ws, block=512):
    R = x.shape[0]
    nw = len(ws)

    def body(*refs):
        xv = refs[0][...]
        for i in range(nw):
            refs[1 + nw + i][...] = _dot(xv, refs[1 + i][...])

    return pl.pallas_call(
        body,
        grid=(R // block,),
        in_specs=[pl.BlockSpec((block, LAT), lambda i: (i, 0))]
        + [pl.BlockSpec((LAT, LAT), lambda i: (0, 0))] * nw,
        out_specs=[pl.BlockSpec((block, LAT), lambda i: (i, 0))] * nw,
        out_shape=[jax.ShapeDtypeStruct((R, LAT), F32)] * nw,
    )(x, *ws)


def _sc_gather_all(tables, idxs2d):
    nj = len(tables)
    epads = [i.shape[0] * 128 for i in idxs2d]
    rows_w = [e // 128 // _NW for e in epads]
    joff = [0]
    for r in rows_w:
        joff.append(joff[-1] + r)
    mesh = plsc.VectorSubcoreMesh(core_axis_name="c", subcore_axis_name="s")

    nts = [t.shape[0] for t in tables]

    @functools.partial(
        pl.kernel, mesh=mesh,
        out_type=[jax.ShapeDtypeStruct((e, LAT), F32) for e in epads],
        scratch_types=[
            pltpu.VMEM_SHARED((NMP, LAT), F32),
            pltpu.VMEM((joff[-1], 128), jnp.int32),
            pltpu.VMEM((256, LAT), F32),
            pltpu.SemaphoreType.DMA,
            pltpu.SemaphoreType.DMA((2,)),
        ],
    )
    def k(*refs):
        trefs = refs[:nj]
        irefs = refs[nj:2 * nj]
        orefs = refs[2 * nj:3 * nj]
        sh_t, idx_all, rows_v, gsem, wsem = refs[3 * nj:]
        s = lax.axis_index("s")
        wid = s * _NC + lax.axis_index("c")
        icps = [pltpu.async_copy(irefs[j].at[pl.ds(wid * rows_w[j], rows_w[j])],
                                 idx_all.at[pl.ds(joff[j], rows_w[j])], gsem)
                for j in range(nj)]
        for cp in icps:
            cp.wait()
        for j in range(nj):
            rpt_t = nts[j] // _NS
            pltpu.sync_copy(trefs[j].at[pl.ds(s * rpt_t, rpt_t)],
                            sh_t.at[pl.ds(s * rpt_t, rpt_t)])
            plsc.subcore_barrier()
            rows = rows_w[j]
            base = wid * rows

            def gbody(sg, _, j=j, base=base):
                h = sg % 2

                @pl.when(sg >= 2)
                def _():
                    pltpu.make_async_copy(
                        rows_v.at[pl.ds(h * 128, 128)],
                        orefs[j].at[pl.ds((base + sg - 2) * 128, 128)],
                        wsem.at[h]).wait()

                pltpu.async_copy(sh_t.at[idx_all.at[joff[j] + sg]],
                                 rows_v.at[pl.ds(h * 128, 128)], gsem).wait()
                pltpu.async_copy(rows_v.at[pl.ds(h * 128, 128)],
                                 orefs[j].at[pl.ds((base + sg) * 128, 128)],
                                 wsem.at[h])
                return 0

            lax.fori_loop(0, rows, gbody, 0)
            for t in range(max(rows - 2, 0), rows):
                pltpu.make_async_copy(
                    rows_v.at[pl.ds((t % 2) * 128, 128)],
                    orefs[j].at[pl.ds((base + t) * 128, 128)],
                    wsem.at[t % 2]).wait()
            plsc.subcore_barrier()

    return k(*tables, *idxs2d)


def _sc_scatter_all(zeros, vals, idxs2d, out_rows):
    nj = len(vals)
    epads = [i.shape[0] * 128 for i in idxs2d]
    mesh = plsc.VectorSubcoreMesh(core_axis_name="c", subcore_axis_name="s")

    rows_w = [e // 128 // _NW for e in epads]
    joff = [0]
    for r in rows_w:
        joff.append(joff[-1] + r)

    @functools.partial(
        pl.kernel, mesh=mesh,
        out_type=[jax.ShapeDtypeStruct((2, r, LAT), F32) for r in out_rows],
        scratch_types=[
            pltpu.VMEM_SHARED((NMP, LAT), F32),
            pltpu.VMEM((joff[-1], 128), jnp.int32),
            pltpu.VMEM((256, LAT), F32),
            pltpu.SemaphoreType.DMA,
            pltpu.SemaphoreType.DMA((2,)),
        ],
    )
    def k(*refs):
        zr = refs[0]
        vrefs = refs[1:1 + nj]
        irefs = refs[1 + nj:1 + 2 * nj]
        orefs = refs[1 + 2 * nj:1 + 3 * nj]
        acc, idx_all, buf, sem, lsem = refs[1 + 3 * nj:]
        c = lax.axis_index("c")
        s = lax.axis_index("s")
        wid = s * _NC + c
        icps = [pltpu.async_copy(irefs[j].at[pl.ds(wid * rows_w[j], rows_w[j])],
                                 idx_all.at[pl.ds(joff[j], rows_w[j])], sem)
                for j in range(nj)]
        for cp in icps:
            cp.wait()
        for j in range(nj):
            rpt = out_rows[j] // _NS
            pltpu.sync_copy(zr, buf.at[pl.ds(0, 128)])
            for r in range(rpt // 128):
                pltpu.sync_copy(buf.at[pl.ds(0, 128)],
                                acc.at[pl.ds(s * rpt + r * 128, 128)])
            plsc.subcore_barrier()
            rows = rows_w[j]
            base = wid * rows
            pltpu.async_copy(vrefs[j].at[pl.ds(base * 128, 128)],
                             buf.at[pl.ds(0, 128)], lsem.at[0])

            def sbody(g, _, j=j, base=base, rows=rows):
                h = g % 2
                pltpu.make_async_copy(
                    vrefs[j].at[pl.ds((base + g) * 128, 128)],
                    buf.at[pl.ds(h * 128, 128)], lsem.at[h]).wait()

                @pl.when(g + 1 < rows)
                def _():
                    pltpu.async_copy(
                        vrefs[j].at[pl.ds((base + g + 1) * 128, 128)],
                        buf.at[pl.ds((1 - h) * 128, 128)], lsem.at[1 - h])

                pltpu.sync_copy(buf.at[pl.ds(h * 128, 128)],
                                acc.at[idx_all.at[joff[j] + g]], add=True)
                return 0

            lax.fori_loop(0, rows, sbody, 0)
            plsc.subcore_barrier()
            for r in range(rpt // 128):
                r0 = s * rpt + r * 128
                pltpu.sync_copy(acc.at[pl.ds(r0, 128)],
                                buf.at[pl.ds(0, 128)])
                pltpu.sync_copy(buf.at[pl.ds(0, 128)],
                                orefs[j].at[c, pl.ds(r0, 128)])
            plsc.subcore_barrier()

    return k(zeros, *vals, *idxs2d)


def _pad_rows(x, rows, fill=0.0):
    return jnp.pad(x, ((0, rows - x.shape[0]), (0, 0)), constant_values=fill)


def _pad_idx(idx, n, fill):
    return jnp.pad(idx, (0, n - idx.shape[0]), constant_values=fill)


def _mlp_params(p):
    (w1, b1), (w2, b2), (w3, b3) = p["layers"]
    return w1, b1, w2, b2, w3, b3


def kernel(mesh_features, obj_features, mesh_kinematic, obj_kinematic,
           index_mm, index_mo, index_om, index_ff, e_mm, e_mo, e_ff, params):
    pm = params
    m_kin = jax.nn.one_hot(mesh_kinematic, 3, dtype=F32)
    o_kin = jax.nn.one_hot(obj_kinematic, 3, dtype=F32)
    m_in = jnp.concatenate([mesh_features, m_kin], axis=-1)
    o_in = jnp.concatenate([obj_features, o_kin], axis=-1)
    m_in = (m_in - pm["node_mean"]) / pm["node_std"]
    o_in = (o_in - pm["node_mean"]) / pm["node_std"]
    nk = jax.random.key(42)
    m_in = m_in.at[:, :3].add(
        1e-05 * jax.random.normal(jax.random.fold_in(nk, 0), (N_MESH, 3), F32))
    o_in = o_in.at[:, :3].add(
        1e-05 * jax.random.normal(jax.random.fold_in(nk, 1), (N_OBJ, 3), F32))
    m_in = _pad_rows(jnp.pad(m_in, ((0, 0), (0, 5))), NMP)
    o_in = _pad_rows(jnp.pad(o_in, ((0, 0), (0, 5))), NOP)

    e_mm_p = _pad_rows(e_mm, E_MM_P)
    e_mo_p = _pad_rows(e_mo, E_MO_P)
    e_ff_p = _pad_rows(jnp.pad(e_ff, ((0, 0), (0, 6))), E_FF_P)

    imm0 = _pad_idx(index_mm[0], E_MM_P, 0)
    imm1 = _pad_idx(index_mm[1], E_MM_P, N_MESH)
    imo0 = _pad_idx(index_mo[0], E_MO_P, 0)
    imo1 = _pad_idx(index_mo[1], E_MO_P, N_OBJ)
    iom0 = _pad_idx(index_om[0], E_MO_P, 0)
    iom1 = _pad_idx(index_om[1], E_MO_P, N_MESH)
    iff0 = _pad_idx(index_ff[0], E_FF_P, 0)
    iff1 = _pad_idx(index_ff[1], E_FF_P, N_MESH)
    i2 = lambda ix: ix.reshape(-1, 128)
    gidx = [i2(imm0), i2(imm1), i2(imo0), i2(imo1),
            i2(iom0), i2(iom1), i2(iff0), i2(iff1)]
    sidx = [i2(imm1), i2(iom1), i2(iff1), i2(imo1)]
    zeros128 = jnp.zeros((128, LAT), F32)

    def fold(enc, mean, std, pad_to):
        w1, b1, w2, b2, w3, b3 = _mlp_params(enc)
        w1f = w1 / std[:, None]
        b1f = b1 - (mean / std) @ w1
        w1f = jnp.pad(w1f, ((0, pad_to - w1f.shape[0]), (0, 0)))
        return w1f, b1f, w2, b2, w3, b3

    w1, b1, w2, b2, w3, b3 = _mlp_params(pm["enc_mesh"])
    m = _mlp([[m_in]], [jnp.pad(w1, ((0, 5), (0, 0)))], b1, w2, b2, w3, b3,
             ln=pm["enc_mesh"]["ln"])
    w1, b1, w2, b2, w3, b3 = _mlp_params(pm["enc_obj"])
    o = _mlp([[o_in]], [jnp.pad(w1, ((0, 5), (0, 0)))], b1, w2, b2, w3, b3,
             ln=pm["enc_obj"]["ln"])
    w1f, b1f, w2, b2, w3, b3 = fold(pm["enc_mm"], pm["edge_mean"], pm["edge_std"], 8)
    lmm = _mlp([[e_mm_p]], [w1f], b1f, w2, b2, w3, b3, ln=pm["enc_mm"]["ln"])
    w1f, b1f, w2, b2, w3, b3 = fold(pm["enc_mo"], pm["edge_mean"], pm["edge_std"], 8)
    lmo = _mlp([[e_mo_p]], [w1f], b1f, w2, b2, w3, b3, ln=pm["enc_mo"]["ln"])
    w1f, b1f, w2, b2, w3, b3 = fold(pm["enc_om"], pm["edge_mean"], pm["edge_std"], 8)
    lom = _mlp([[e_mo_p]], [w1f], b1f, w2, b2, w3, b3, ln=pm["enc_om"]["ln"])
    w1f, b1f, w2, b2, w3, b3 = fold(pm["enc_ff"], pm["face_mean"], pm["face_std"], 40)
    lff = _mlp([[e_ff_p]], [w1f], b1f, w2, b2, w3, b3, ln=pm["enc_ff"]["ln"])

    for sp in pm["steps"]:
        wmm = _mlp_params(sp["mm"])
        wmo = _mlp_params(sp["mo"])
        wom = _mlp_params(sp["om"])
        wff = _mlp_params(sp["ff"])
        tm = _tables(m, [wmm[0][:128], wmm[0][128:256], wmo[0][:128],
                         wom[0][128:256], wff[0][:128], wff[0][128:256]])
        to = _tables(o, [wmo[0][128:256], wom[0][:128]])
        g = _sc_gather_all(
            [tm[0], tm[1], tm[2], to[0], to[1], tm[3], tm[4], tm[5]], gidx)
        lmm = _mlp([[g[0], g[1]], [lmm]], [None, wmm[0][256:]], wmm[1],
                   wmm[2], wmm[3], wmm[4], wmm[5],
                   ln=sp["mm"]["ln"], res=lmm)
        lmo = _mlp([[g[2], g[3]], [lmo]], [None, wmo[0][256:]], wmo[1],
                   wmo[2], wmo[3], wmo[4], wmo[5],
                   ln=sp["mo"]["ln"], res=lmo)
        lom = _mlp([[g[4], g[5]], [lom]], [None, wom[0][256:]], wom[1],
                   wom[2], wom[3], wom[4], wom[5],
                   ln=sp["om"]["ln"], res=lom)
        lff = _mlp([[g[6], g[7]], [lff]], [None, wff[0][256:]], wff[1],
                   wff[2], wff[3], wff[4], wff[5],
                   ln=sp["ff"]["ln"], res=lff)
        pmm, pom, pff, pmo = _sc_scatter_all(
            zeros128, [lmm, lom, lff, lmo], sidx, [NMP, NMP, NMP, NOP])
        wn = _mlp_params(sp["mesh_node"])
        m = _mlp([[m], [pmm[0], pmm[1]], [pom[0], pom[1]], [pff[0], pff[1]]],
                 [wn[0][:128], wn[0][128:256], wn[0][256:384], wn[0][384:]],
                 wn[1], wn[2], wn[3], wn[4], wn[5],
                 ln=sp["mesh_node"]["ln"], res=m)
        wo = _mlp_params(sp["obj_node"])
        o = _mlp([[o], [pmo[0], pmo[1]]], [wo[0][:128], wo[0][128:]],
                 wo[1], wo[2], wo[3], wo[4], wo[5],
                 ln=sp["obj_node"]["ln"], res=o)

    w1, b1, w2, b2, w3, b3 = _mlp_params(pm["dec_mesh"])
    md = _mlp([[m]], [w1], b1, w2, b2,
              jnp.pad(w3, ((0, 0), (0, 125))), jnp.pad(b3, (0, 125)))
    w1, b1, w2, b2, w3, b3 = _mlp_params(pm["dec_obj"])
    od = _mlp([[o]], [w1], b1, w2, b2,
              jnp.pad(w3, ((0, 0), (0, 125))), jnp.pad(b3, (0, 125)))
    return md[:N_MESH, :3], od[:N_OBJ, :3]


# speedup vs baseline: 2.5114x; 1.2550x over previous
import functools

import jax
import jax.numpy as jnp
from jax import lax
from jax.experimental import pallas as pl
from jax.experimental.pallas import tpu as pltpu
from jax.experimental.pallas import tpu_sc as plsc

F32 = jnp.float32
LAT = 128
N_MESH, N_OBJ = 10000, 2000
NMP, NOP = 10240, 2048
E_MM_P, E_MO_P, E_FF_P = 163840, 16384, 8192
_NC, _NS, _NW = 2, 16, 32


def _dot(a, b):
    return lax.dot_general(a, b, (((1,), (0,)), ((), ())),
                           preferred_element_type=F32,
                           precision=lax.Precision.DEFAULT)


def _mlp(groups, w1s, b1, w2, b2, w3, b3, ln=None, res=None, block=1024):
    xs = [x for g in groups for x in g]
    R = xs[0].shape[0]
    sizes = [len(g) for g in groups]
    has_w = [w is not None for w in w1s]
    ws = [w for w in w1s if w is not None]
    nx, nw = len(xs), len(ws)

    def body(*refs):
        xr = refs[:nx]
        wr = refs[nx:nx + nw]
        b1r, w2r, b2r, w3r, b3r = refs[nx + nw:nx + nw + 5]
        p = nx + nw + 5
        if ln is not None:
            lgr, lbr = refs[p], refs[p + 1]
            p += 2
        if res is not None:
            rr = refs[p]
            p += 1
        out = refs[-1]
        h = None
        k = wi = 0
        for gi, sz in enumerate(sizes):
            acc = xr[k][...]
            for j in range(1, sz):
                acc = acc + xr[k + j][...]
            k += sz
            if has_w[gi]:
                acc = _dot(acc, wr[wi][...])
                wi += 1
            h = acc if h is None else h + acc
        h = jnp.maximum(h + b1r[...], 0.0)
        h = jnp.maximum(_dot(h, w2r[...]) + b2r[...], 0.0)
        y = _dot(h, w3r[...]) + b3r[...]
        if ln is not None:
            mu = jnp.mean(y, axis=-1, keepdims=True)
            var = jnp.mean((y - mu) ** 2, axis=-1, keepdims=True)
            y = (y - mu) * lax.rsqrt(var + 1e-5) * lgr[...] + lbr[...]
        if res is not None:
            y = y + rr[...]
        out[...] = y

    args = list(xs) + list(ws) + [b1.reshape(1, -1), w2, b2.reshape(1, -1),
                                  w3, b3.reshape(1, -1)]
    if ln is not None:
        args += [ln[0].reshape(1, -1), ln[1].reshape(1, -1)]
    if res is not None:
        args.append(res)
    in_specs = [pl.BlockSpec((block, x.shape[1]), lambda i: (i, 0)) for x in xs]
    in_specs += [pl.BlockSpec(w.shape, lambda i: (0, 0)) for w in ws]
    in_specs += [pl.BlockSpec(a.shape, lambda i: (0, 0)) for a in args[nx + nw:nx + nw + 5]]
    if ln is not None:
        in_specs += [pl.BlockSpec((1, LAT), lambda i: (0, 0))] * 2
    if res is not None:
        in_specs.append(pl.BlockSpec((block, LAT), lambda i: (i, 0)))
    return pl.pallas_call(
        body,
        grid=(R // block,),
        in_specs=in_specs,
        out_specs=pl.BlockSpec((block, LAT), lambda i: (i, 0)),
        out_shape=jax.ShapeDtypeStruct((R, LAT), F32),
    )(*args)


def _tables(x, ws, block=1024):
    R = x.shape[0]
    nw = len(ws)

    def body(*refs):
        xv = refs[0][...]
        for i in range(nw):
            refs[1 + nw + i][...] = _dot(xv, refs[1 + i][...])

    return pl.pallas_call(
        body,
        grid=(R // block,),
        in_specs=[pl.BlockSpec((block, LAT), lambda i: (i, 0))]
        + [pl.BlockSpec((LAT, LAT), lambda i: (0, 0))] * nw,
        out_specs=[pl.BlockSpec((block, LAT), lambda i: (i, 0))] * nw,
        out_shape=[jax.ShapeDtypeStruct((R, LAT), F32)] * nw,
    )(x, *ws)


def _sc_gather_all(tables, idxs2d):
    nj = len(tables)
    epads = [i.shape[0] * 128 for i in idxs2d]
    rows_w = [e // 128 // _NW for e in epads]
    joff = [0]
    for r in rows_w:
        joff.append(joff[-1] + r)
    mesh = plsc.VectorSubcoreMesh(core_axis_name="c", subcore_axis_name="s")

    nts = [t.shape[0] for t in tables]

    @functools.partial(
        pl.kernel, mesh=mesh,
        out_type=[jax.ShapeDtypeStruct((e, LAT), F32) for e in epads],
        scratch_types=[
            pltpu.VMEM_SHARED((NMP, LAT), F32),
            pltpu.VMEM((joff[-1], 128), jnp.int32),
            pltpu.VMEM((256, LAT), F32),
            pltpu.SemaphoreType.DMA,
            pltpu.SemaphoreType.DMA((2,)),
        ],
    )
    def k(*refs):
        trefs = refs[:nj]
        irefs = refs[nj:2 * nj]
        orefs = refs[2 * nj:3 * nj]
        sh_t, idx_all, rows_v, gsem, wsem = refs[3 * nj:]
        s = lax.axis_index("s")
        wid = s * _NC + lax.axis_index("c")
        icps = [pltpu.async_copy(irefs[j].at[pl.ds(wid * rows_w[j], rows_w[j])],
                                 idx_all.at[pl.ds(joff[j], rows_w[j])], gsem)
                for j in range(nj)]
        for cp in icps:
            cp.wait()
        for j in range(nj):
            rpt_t = nts[j] // _NS
            pltpu.sync_copy(trefs[j].at[pl.ds(s * rpt_t, rpt_t)],
                            sh_t.at[pl.ds(s * rpt_t, rpt_t)])
            plsc.subcore_barrier()
            rows = rows_w[j]
            base = wid * rows

            def gbody(sg, _, j=j, base=base):
                h = sg % 2

                @pl.when(sg >= 2)
                def _():
                    pltpu.make_async_copy(
                        rows_v.at[pl.ds(h * 128, 128)],
                        orefs[j].at[pl.ds((base + sg - 2) * 128, 128)],
                        wsem.at[h]).wait()

                pltpu.async_copy(sh_t.at[idx_all.at[joff[j] + sg]],
                                 rows_v.at[pl.ds(h * 128, 128)], gsem).wait()
                pltpu.async_copy(rows_v.at[pl.ds(h * 128, 128)],
                                 orefs[j].at[pl.ds((base + sg) * 128, 128)],
                                 wsem.at[h])
                return 0

            lax.fori_loop(0, rows, gbody, 0)
            for t in range(max(rows - 2, 0), rows):
                pltpu.make_async_copy(
                    rows_v.at[pl.ds((t % 2) * 128, 128)],
                    orefs[j].at[pl.ds((base + t) * 128, 128)],
                    wsem.at[t % 2]).wait()
            plsc.subcore_barrier()

    return k(*tables, *idxs2d)


def _sc_scatter_all(zeros, vals, idxs2d, out_rows):
    nj = len(vals)
    epads = [i.shape[0] * 128 for i in idxs2d]
    mesh = plsc.VectorSubcoreMesh(core_axis_name="c", subcore_axis_name="s")

    rows_w = [e // 128 // _NW for e in epads]
    joff = [0]
    for r in rows_w:
        joff.append(joff[-1] + r)

    @functools.partial(
        pl.kernel, mesh=mesh,
        out_type=[jax.ShapeDtypeStruct((2, r, LAT), F32) for r in out_rows],
        scratch_types=[
            pltpu.VMEM_SHARED((NMP, LAT), F32),
            pltpu.VMEM((joff[-1], 128), jnp.int32),
            pltpu.VMEM((256, LAT), F32),
            pltpu.SemaphoreType.DMA,
            pltpu.SemaphoreType.DMA((2,)),
        ],
    )
    def k(*refs):
        zr = refs[0]
        vrefs = refs[1:1 + nj]
        irefs = refs[1 + nj:1 + 2 * nj]
        orefs = refs[1 + 2 * nj:1 + 3 * nj]
        acc, idx_all, buf, sem, lsem = refs[1 + 3 * nj:]
        c = lax.axis_index("c")
        s = lax.axis_index("s")
        wid = s * _NC + c
        icps = [pltpu.async_copy(irefs[j].at[pl.ds(wid * rows_w[j], rows_w[j])],
                                 idx_all.at[pl.ds(joff[j], rows_w[j])], sem)
                for j in range(nj)]
        for cp in icps:
            cp.wait()
        for j in range(nj):
            rpt = out_rows[j] // _NS
            pltpu.sync_copy(zr, buf)
            off = 0
            while off < rpt:
                step = 256 if rpt - off >= 256 else 128
                pltpu.sync_copy(buf.at[pl.ds(0, step)],
                                acc.at[pl.ds(s * rpt + off, step)])
                off += step
            plsc.subcore_barrier()
            rows = rows_w[j]
            base = wid * rows
            pltpu.async_copy(vrefs[j].at[pl.ds(base * 128, 128)],
                             buf.at[pl.ds(0, 128)], lsem.at[0])

            def sbody(g, _, j=j, base=base, rows=rows):
                h = g % 2
                pltpu.make_async_copy(
                    vrefs[j].at[pl.ds((base + g) * 128, 128)],
                    buf.at[pl.ds(h * 128, 128)], lsem.at[h]).wait()

                @pl.when(g + 1 < rows)
                def _():
                    pltpu.async_copy(
                        vrefs[j].at[pl.ds((base + g + 1) * 128, 128)],
                        buf.at[pl.ds((1 - h) * 128, 128)], lsem.at[1 - h])

                pltpu.sync_copy(buf.at[pl.ds(h * 128, 128)],
                                acc.at[idx_all.at[joff[j] + g]], add=True)
                return 0

            lax.fori_loop(0, rows, sbody, 0)
            plsc.subcore_barrier()
            pltpu.sync_copy(acc.at[pl.ds(s * rpt, rpt)],
                            orefs[j].at[c, pl.ds(s * rpt, rpt)])
            plsc.subcore_barrier()

    return k(zeros, *vals, *idxs2d)


def _pad_rows(x, rows, fill=0.0):
    return jnp.pad(x, ((0, rows - x.shape[0]), (0, 0)), constant_values=fill)


def _pad_idx(idx, n, fill):
    return jnp.pad(idx, (0, n - idx.shape[0]), constant_values=fill)


def _mlp_params(p):
    (w1, b1), (w2, b2), (w3, b3) = p["layers"]
    return w1, b1, w2, b2, w3, b3


def kernel(mesh_features, obj_features, mesh_kinematic, obj_kinematic,
           index_mm, index_mo, index_om, index_ff, e_mm, e_mo, e_ff, params):
    pm = params
    m_kin = jax.nn.one_hot(mesh_kinematic, 3, dtype=F32)
    o_kin = jax.nn.one_hot(obj_kinematic, 3, dtype=F32)
    m_in = jnp.concatenate([mesh_features, m_kin], axis=-1)
    o_in = jnp.concatenate([obj_features, o_kin], axis=-1)
    m_in = (m_in - pm["node_mean"]) / pm["node_std"]
    o_in = (o_in - pm["node_mean"]) / pm["node_std"]
    nk = jax.random.key(42)
    m_in = m_in.at[:, :3].add(
        1e-05 * jax.random.normal(jax.random.fold_in(nk, 0), (N_MESH, 3), F32))
    o_in = o_in.at[:, :3].add(
        1e-05 * jax.random.normal(jax.random.fold_in(nk, 1), (N_OBJ, 3), F32))
    m_in = _pad_rows(jnp.pad(m_in, ((0, 0), (0, 5))), NMP)
    o_in = _pad_rows(jnp.pad(o_in, ((0, 0), (0, 5))), NOP)

    e_mm_p = _pad_rows(e_mm, E_MM_P)
    e_mo_p = _pad_rows(e_mo, E_MO_P)
    e_ff_p = _pad_rows(jnp.pad(e_ff, ((0, 0), (0, 6))), E_FF_P)

    imm0 = _pad_idx(index_mm[0], E_MM_P, 0)
    imm1 = _pad_idx(index_mm[1], E_MM_P, N_MESH)
    imo0 = _pad_idx(index_mo[0], E_MO_P, 0)
    imo1 = _pad_idx(index_mo[1], E_MO_P, N_OBJ)
    iom0 = _pad_idx(index_om[0], E_MO_P, 0)
    iom1 = _pad_idx(index_om[1], E_MO_P, N_MESH)
    iff0 = _pad_idx(index_ff[0], E_FF_P, 0)
    iff1 = _pad_idx(index_ff[1], E_FF_P, N_MESH)
    i2 = lambda ix: ix.reshape(-1, 128)
    gidx = [i2(imm0), i2(imm1), i2(imo0), i2(imo1),
            i2(iom0), i2(iom1), i2(iff0), i2(iff1)]
    sidx = [i2(imm1), i2(iom1), i2(iff1), i2(imo1)]
    zeros256 = jnp.zeros((256, LAT), F32)

    def fold(enc, mean, std, pad_to):
        w1, b1, w2, b2, w3, b3 = _mlp_params(enc)
        w1f = w1 / std[:, None]
        b1f = b1 - (mean / std) @ w1
        w1f = jnp.pad(w1f, ((0, pad_to - w1f.shape[0]), (0, 0)))
        return w1f, b1f, w2, b2, w3, b3

    w1, b1, w2, b2, w3, b3 = _mlp_params(pm["enc_mesh"])
    m = _mlp([[m_in]], [jnp.pad(w1, ((0, 5), (0, 0)))], b1, w2, b2, w3, b3,
             ln=pm["enc_mesh"]["ln"])
    w1, b1, w2, b2, w3, b3 = _mlp_params(pm["enc_obj"])
    o = _mlp([[o_in]], [jnp.pad(w1, ((0, 5), (0, 0)))], b1, w2, b2, w3, b3,
             ln=pm["enc_obj"]["ln"])
    w1f, b1f, w2, b2, w3, b3 = fold(pm["enc_mm"], pm["edge_mean"], pm["edge_std"], 8)
    lmm = _mlp([[e_mm_p]], [w1f], b1f, w2, b2, w3, b3, ln=pm["enc_mm"]["ln"])
    w1f, b1f, w2, b2, w3, b3 = fold(pm["enc_mo"], pm["edge_mean"], pm["edge_std"], 8)
    lmo = _mlp([[e_mo_p]], [w1f], b1f, w2, b2, w3, b3, ln=pm["enc_mo"]["ln"])
    w1f, b1f, w2, b2, w3, b3 = fold(pm["enc_om"], pm["edge_mean"], pm["edge_std"], 8)
    lom = _mlp([[e_mo_p]], [w1f], b1f, w2, b2, w3, b3, ln=pm["enc_om"]["ln"])
    w1f, b1f, w2, b2, w3, b3 = fold(pm["enc_ff"], pm["face_mean"], pm["face_std"], 40)
    lff = _mlp([[e_ff_p]], [w1f], b1f, w2, b2, w3, b3, ln=pm["enc_ff"]["ln"])

    for sp in pm["steps"]:
        wmm = _mlp_params(sp["mm"])
        wmo = _mlp_params(sp["mo"])
        wom = _mlp_params(sp["om"])
        wff = _mlp_params(sp["ff"])
        tm = _tables(m, [wmm[0][:128], wmm[0][128:256], wmo[0][:128],
                         wom[0][128:256], wff[0][:128], wff[0][128:256]])
        to = _tables(o, [wmo[0][128:256], wom[0][:128]])
        g = _sc_gather_all(
            [tm[0], tm[1], tm[2], to[0], to[1], tm[3], tm[4], tm[5]], gidx)
        lmm = _mlp([[g[0], g[1]], [lmm]], [None, wmm[0][256:]], wmm[1],
                   wmm[2], wmm[3], wmm[4], wmm[5],
                   ln=sp["mm"]["ln"], res=lmm)
        lmo = _mlp([[g[2], g[3]], [lmo]], [None, wmo[0][256:]], wmo[1],
                   wmo[2], wmo[3], wmo[4], wmo[5],
                   ln=sp["mo"]["ln"], res=lmo)
        lom = _mlp([[g[4], g[5]], [lom]], [None, wom[0][256:]], wom[1],
                   wom[2], wom[3], wom[4], wom[5],
                   ln=sp["om"]["ln"], res=lom)
        lff = _mlp([[g[6], g[7]], [lff]], [None, wff[0][256:]], wff[1],
                   wff[2], wff[3], wff[4], wff[5],
                   ln=sp["ff"]["ln"], res=lff)
        pmm, pom, pff, pmo = _sc_scatter_all(
            zeros256, [lmm, lom, lff, lmo], sidx, [NMP, NMP, NMP, NOP])
        wn = _mlp_params(sp["mesh_node"])
        m = _mlp([[m], [pmm[0], pmm[1]], [pom[0], pom[1]], [pff[0], pff[1]]],
                 [wn[0][:128], wn[0][128:256], wn[0][256:384], wn[0][384:]],
                 wn[1], wn[2], wn[3], wn[4], wn[5],
                 ln=sp["mesh_node"]["ln"], res=m)
        wo = _mlp_params(sp["obj_node"])
        o = _mlp([[o], [pmo[0], pmo[1]]], [wo[0][:128], wo[0][128:]],
                 wo[1], wo[2], wo[3], wo[4], wo[5],
                 ln=sp["obj_node"]["ln"], res=o)

    w1, b1, w2, b2, w3, b3 = _mlp_params(pm["dec_mesh"])
    md = _mlp([[m]], [w1], b1, w2, b2,
              jnp.pad(w3, ((0, 0), (0, 125))), jnp.pad(b3, (0, 125)))
    w1, b1, w2, b2, w3, b3 = _mlp_params(pm["dec_obj"])
    od = _mlp([[o]], [w1], b1, w2, b2,
              jnp.pad(w3, ((0, 0), (0, 125))), jnp.pad(b3, (0, 125)))
    return md[:N_MESH, :3], od[:N_OBJ, :3]


# speedup vs baseline: 2.9164x; 1.1613x over previous
import functools

import jax
import jax.numpy as jnp
from jax import lax
from jax.experimental import pallas as pl
from jax.experimental.pallas import tpu as pltpu
from jax.experimental.pallas import tpu_sc as plsc

F32 = jnp.float32
LAT = 128
N_MESH, N_OBJ = 10000, 2000
NMP, NOP = 10240, 2048
E_MM_P, E_MO_P, E_FF_P = 163840, 16384, 8192
_NC, _NS, _NW = 2, 16, 32


def _dot(a, b):
    return lax.dot_general(a, b, (((1,), (0,)), ((), ())),
                           preferred_element_type=F32,
                           precision=lax.Precision.DEFAULT)


def _mlp(groups, w1s, b1, w2, b2, w3, b3, ln=None, res=None, block=2048):
    xs = [x for g in groups for x in g]
    R = xs[0].shape[0]
    sizes = [len(g) for g in groups]
    has_w = [w is not None for w in w1s]
    ws = [w for w in w1s if w is not None]
    nx, nw = len(xs), len(ws)

    def body(*refs):
        xr = refs[:nx]
        wr = refs[nx:nx + nw]
        b1r, w2r, b2r, w3r, b3r = refs[nx + nw:nx + nw + 5]
        p = nx + nw + 5
        if ln is not None:
            lgr, lbr = refs[p], refs[p + 1]
            p += 2
        if res is not None:
            rr = refs[p]
            p += 1
        out = refs[-1]
        h = None
        k = wi = 0
        for gi, sz in enumerate(sizes):
            acc = xr[k][...]
            for j in range(1, sz):
                acc = acc + xr[k + j][...]
            k += sz
            if has_w[gi]:
                acc = _dot(acc, wr[wi][...])
                wi += 1
            h = acc if h is None else h + acc
        h = jnp.maximum(h + b1r[...], 0.0)
        h = jnp.maximum(_dot(h, w2r[...]) + b2r[...], 0.0)
        y = _dot(h, w3r[...]) + b3r[...]
        if ln is not None:
            mu = jnp.mean(y, axis=-1, keepdims=True)
            var = jnp.mean((y - mu) ** 2, axis=-1, keepdims=True)
            y = (y - mu) * lax.rsqrt(var + 1e-5) * lgr[...] + lbr[...]
        if res is not None:
            y = y + rr[...]
        out[...] = y

    args = list(xs) + list(ws) + [b1.reshape(1, -1), w2, b2.reshape(1, -1),
                                  w3, b3.reshape(1, -1)]
    if ln is not None:
        args += [ln[0].reshape(1, -1), ln[1].reshape(1, -1)]
    if res is not None:
        args.append(res)
    in_specs = [pl.BlockSpec((block, x.shape[1]), lambda i: (i, 0)) for x in xs]
    in_specs += [pl.BlockSpec(w.shape, lambda i: (0, 0)) for w in ws]
    in_specs += [pl.BlockSpec(a.shape, lambda i: (0, 0)) for a in args[nx + nw:nx + nw + 5]]
    if ln is not None:
        in_specs += [pl.BlockSpec((1, LAT), lambda i: (0, 0))] * 2
    if res is not None:
        in_specs.append(pl.BlockSpec((block, LAT), lambda i: (i, 0)))
    return pl.pallas_call(
        body,
        grid=(R // block,),
        in_specs=in_specs,
        out_specs=pl.BlockSpec((block, LAT), lambda i: (i, 0)),
        out_shape=jax.ShapeDtypeStruct((R, LAT), F32),
    )(*args)


def _tables(x, ws, block=2048):
    R = x.shape[0]
    nw = len(ws)

    def body(*refs):
        xv = refs[0][...]
        for i in range(nw):
            refs[1 + nw + i][...] = _dot(xv, refs[1 + i][...])

    return pl.pallas_call(
        body,
        grid=(R // block,),
        in_specs=[pl.BlockSpec((block, LAT), lambda i: (i, 0))]
        + [pl.BlockSpec((LAT, LAT), lambda i: (0, 0))] * nw,
        out_specs=[pl.BlockSpec((block, LAT), lambda i: (i, 0))] * nw,
        out_shape=[jax.ShapeDtypeStruct((R, LAT), F32)] * nw,
    )(x, *ws)


def _sc_gather_all(tables, idxs2d):
    nj = len(tables)
    epads = [i.shape[0] * 128 for i in idxs2d]
    rows_w = [e // 128 // _NW for e in epads]
    joff = [0]
    for r in rows_w:
        joff.append(joff[-1] + r)
    mesh = plsc.VectorSubcoreMesh(core_axis_name="c", subcore_axis_name="s")

    nts = [t.shape[0] for t in tables]

    @functools.partial(
        pl.kernel, mesh=mesh,
        out_type=[jax.ShapeDtypeStruct((e, LAT), F32) for e in epads],
        scratch_types=[
            pltpu.VMEM_SHARED((NMP, LAT), F32),
            pltpu.VMEM((joff[-1], 128), jnp.int32),
            pltpu.VMEM((256, LAT), F32),
            pltpu.SemaphoreType.DMA,
            pltpu.SemaphoreType.DMA((2,)),
        ],
    )
    def k(*refs):
        trefs = refs[:nj]
        irefs = refs[nj:2 * nj]
        orefs = refs[2 * nj:3 * nj]
        sh_t, idx_all, rows_v, gsem, wsem = refs[3 * nj:]
        s = lax.axis_index("s")
        wid = s * _NC + lax.axis_index("c")
        icps = [pltpu.async_copy(irefs[j].at[pl.ds(wid * rows_w[j], rows_w[j])],
                                 idx_all.at[pl.ds(joff[j], rows_w[j])], gsem)
                for j in range(nj)]
        for cp in icps:
            cp.wait()
        for j in range(nj):
            rpt_t = nts[j] // _NS
            pltpu.sync_copy(trefs[j].at[pl.ds(s * rpt_t, rpt_t)],
                            sh_t.at[pl.ds(s * rpt_t, rpt_t)])
            plsc.subcore_barrier()
            rows = rows_w[j]
            base = wid * rows

            def gbody(sg, _, j=j, base=base):
                h = sg % 2

                @pl.when(sg >= 2)
                def _():
                    pltpu.make_async_copy(
                        rows_v.at[pl.ds(h * 128, 128)],
                        orefs[j].at[pl.ds((base + sg - 2) * 128, 128)],
                        wsem.at[h]).wait()

                pltpu.async_copy(sh_t.at[idx_all.at[joff[j] + sg]],
                                 rows_v.at[pl.ds(h * 128, 128)], gsem).wait()
                pltpu.async_copy(rows_v.at[pl.ds(h * 128, 128)],
                                 orefs[j].at[pl.ds((base + sg) * 128, 128)],
                                 wsem.at[h])
                return 0

            lax.fori_loop(0, rows, gbody, 0)
            for t in range(max(rows - 2, 0), rows):
                pltpu.make_async_copy(
                    rows_v.at[pl.ds((t % 2) * 128, 128)],
                    orefs[j].at[pl.ds((base + t) * 128, 128)],
                    wsem.at[t % 2]).wait()
            plsc.subcore_barrier()

    return k(*tables, *idxs2d)


def _sc_scatter_all(zeros, vals, idxs2d, out_rows):
    nj = len(vals)
    epads = [i.shape[0] * 128 for i in idxs2d]
    mesh = plsc.VectorSubcoreMesh(core_axis_name="c", subcore_axis_name="s")

    rows_w = [e // 128 // _NW for e in epads]
    joff = [0]
    for r in rows_w:
        joff.append(joff[-1] + r)

    @functools.partial(
        pl.kernel, mesh=mesh,
        out_type=[jax.ShapeDtypeStruct((2, r, LAT), F32) for r in out_rows],
        scratch_types=[
            pltpu.VMEM_SHARED((NMP, LAT), F32),
            pltpu.VMEM((joff[-1], 128), jnp.int32),
            pltpu.VMEM((256, LAT), F32),
            pltpu.SemaphoreType.DMA,
            pltpu.SemaphoreType.DMA((2,)),
        ],
    )
    def k(*refs):
        zr = refs[0]
        vrefs = refs[1:1 + nj]
        irefs = refs[1 + nj:1 + 2 * nj]
        orefs = refs[1 + 2 * nj:1 + 3 * nj]
        acc, idx_all, buf, sem, lsem = refs[1 + 3 * nj:]
        c = lax.axis_index("c")
        s = lax.axis_index("s")
        wid = s * _NC + c
        icps = [pltpu.async_copy(irefs[j].at[pl.ds(wid * rows_w[j], rows_w[j])],
                                 idx_all.at[pl.ds(joff[j], rows_w[j])], sem)
                for j in range(nj)]
        for cp in icps:
            cp.wait()
        for j in range(nj):
            rpt = out_rows[j] // _NS
            pltpu.sync_copy(zr, buf)
            off = 0
            while off < rpt:
                step = 256 if rpt - off >= 256 else 128
                pltpu.sync_copy(buf.at[pl.ds(0, step)],
                                acc.at[pl.ds(s * rpt + off, step)])
                off += step
            plsc.subcore_barrier()
            rows = rows_w[j]
            base = wid * rows
            pltpu.async_copy(vrefs[j].at[pl.ds(base * 128, 128)],
                             buf.at[pl.ds(0, 128)], lsem.at[0])

            def sbody(g, _, j=j, base=base, rows=rows):
                h = g % 2
                pltpu.make_async_copy(
                    vrefs[j].at[pl.ds((base + g) * 128, 128)],
                    buf.at[pl.ds(h * 128, 128)], lsem.at[h]).wait()

                @pl.when(g + 1 < rows)
                def _():
                    pltpu.async_copy(
                        vrefs[j].at[pl.ds((base + g + 1) * 128, 128)],
                        buf.at[pl.ds((1 - h) * 128, 128)], lsem.at[1 - h])

                pltpu.sync_copy(buf.at[pl.ds(h * 128, 128)],
                                acc.at[idx_all.at[joff[j] + g]], add=True)
                return 0

            lax.fori_loop(0, rows, sbody, 0)
            plsc.subcore_barrier()
            pltpu.sync_copy(acc.at[pl.ds(s * rpt, rpt)],
                            orefs[j].at[c, pl.ds(s * rpt, rpt)])
            plsc.subcore_barrier()

    return k(zeros, *vals, *idxs2d)


def _pad_rows(x, rows, fill=0.0):
    return jnp.pad(x, ((0, rows - x.shape[0]), (0, 0)), constant_values=fill)


def _pad_idx(idx, n, fill):
    return jnp.pad(idx, (0, n - idx.shape[0]), constant_values=fill)


def _mlp_params(p):
    (w1, b1), (w2, b2), (w3, b3) = p["layers"]
    return w1, b1, w2, b2, w3, b3


def kernel(mesh_features, obj_features, mesh_kinematic, obj_kinematic,
           index_mm, index_mo, index_om, index_ff, e_mm, e_mo, e_ff, params):
    pm = params
    m_kin = jax.nn.one_hot(mesh_kinematic, 3, dtype=F32)
    o_kin = jax.nn.one_hot(obj_kinematic, 3, dtype=F32)
    m_in = jnp.concatenate([mesh_features, m_kin], axis=-1)
    o_in = jnp.concatenate([obj_features, o_kin], axis=-1)
    m_in = (m_in - pm["node_mean"]) / pm["node_std"]
    o_in = (o_in - pm["node_mean"]) / pm["node_std"]
    nk = jax.random.key(42)
    m_in = m_in.at[:, :3].add(
        1e-05 * jax.random.normal(jax.random.fold_in(nk, 0), (N_MESH, 3), F32))
    o_in = o_in.at[:, :3].add(
        1e-05 * jax.random.normal(jax.random.fold_in(nk, 1), (N_OBJ, 3), F32))
    m_in = _pad_rows(jnp.pad(m_in, ((0, 0), (0, 5))), NMP)
    o_in = _pad_rows(jnp.pad(o_in, ((0, 0), (0, 5))), NOP)

    e_mm_p = _pad_rows(e_mm, E_MM_P)
    e_mo_p = _pad_rows(e_mo, E_MO_P)
    e_ff_p = _pad_rows(jnp.pad(e_ff, ((0, 0), (0, 6))), E_FF_P)

    imm0 = _pad_idx(index_mm[0], E_MM_P, 0)
    imm1 = _pad_idx(index_mm[1], E_MM_P, N_MESH)
    imo0 = _pad_idx(index_mo[0], E_MO_P, 0)
    imo1 = _pad_idx(index_mo[1], E_MO_P, N_OBJ)
    iom0 = _pad_idx(index_om[0], E_MO_P, 0)
    iom1 = _pad_idx(index_om[1], E_MO_P, N_MESH)
    iff0 = _pad_idx(index_ff[0], E_FF_P, 0)
    iff1 = _pad_idx(index_ff[1], E_FF_P, N_MESH)
    i2 = lambda ix: ix.reshape(-1, 128)
    gidx = [i2(imm0), i2(imm1), i2(imo0), i2(imo1),
            i2(iom0), i2(iom1), i2(iff0), i2(iff1)]
    sidx = [i2(imm1), i2(iom1), i2(iff1), i2(imo1)]
    zeros256 = jnp.zeros((256, LAT), F32)

    def fold(enc, mean, std, pad_to):
        w1, b1, w2, b2, w3, b3 = _mlp_params(enc)
        w1f = w1 / std[:, None]
        b1f = b1 - (mean / std) @ w1
        w1f = jnp.pad(w1f, ((0, pad_to - w1f.shape[0]), (0, 0)))
        return w1f, b1f, w2, b2, w3, b3

    w1, b1, w2, b2, w3, b3 = _mlp_params(pm["enc_mesh"])
    m = _mlp([[m_in]], [jnp.pad(w1, ((0, 5), (0, 0)))], b1, w2, b2, w3, b3,
             ln=pm["enc_mesh"]["ln"])
    w1, b1, w2, b2, w3, b3 = _mlp_params(pm["enc_obj"])
    o = _mlp([[o_in]], [jnp.pad(w1, ((0, 5), (0, 0)))], b1, w2, b2, w3, b3,
             ln=pm["enc_obj"]["ln"])
    w1f, b1f, w2, b2, w3, b3 = fold(pm["enc_mm"], pm["edge_mean"], pm["edge_std"], 8)
    lmm = _mlp([[e_mm_p]], [w1f], b1f, w2, b2, w3, b3, ln=pm["enc_mm"]["ln"])
    w1f, b1f, w2, b2, w3, b3 = fold(pm["enc_mo"], pm["edge_mean"], pm["edge_std"], 8)
    lmo = _mlp([[e_mo_p]], [w1f], b1f, w2, b2, w3, b3, ln=pm["enc_mo"]["ln"])
    w1f, b1f, w2, b2, w3, b3 = fold(pm["enc_om"], pm["edge_mean"], pm["edge_std"], 8)
    lom = _mlp([[e_mo_p]], [w1f], b1f, w2, b2, w3, b3, ln=pm["enc_om"]["ln"])
    w1f, b1f, w2, b2, w3, b3 = fold(pm["enc_ff"], pm["face_mean"], pm["face_std"], 40)
    lff = _mlp([[e_ff_p]], [w1f], b1f, w2, b2, w3, b3, ln=pm["enc_ff"]["ln"])

    for sp in pm["steps"]:
        wmm = _mlp_params(sp["mm"])
        wmo = _mlp_params(sp["mo"])
        wom = _mlp_params(sp["om"])
        wff = _mlp_params(sp["ff"])
        tm = _tables(m, [wmm[0][:128], wmm[0][128:256], wmo[0][:128],
                         wom[0][128:256], wff[0][:128], wff[0][128:256]])
        to = _tables(o, [wmo[0][128:256], wom[0][:128]])
        g = _sc_gather_all(
            [tm[0], tm[1], tm[2], to[0], to[1], tm[3], tm[4], tm[5]], gidx)
        lmm = _mlp([[g[0], g[1]], [lmm]], [None, wmm[0][256:]], wmm[1],
                   wmm[2], wmm[3], wmm[4], wmm[5],
                   ln=sp["mm"]["ln"], res=lmm)
        lmo = _mlp([[g[2], g[3]], [lmo]], [None, wmo[0][256:]], wmo[1],
                   wmo[2], wmo[3], wmo[4], wmo[5],
                   ln=sp["mo"]["ln"], res=lmo)
        lom = _mlp([[g[4], g[5]], [lom]], [None, wom[0][256:]], wom[1],
                   wom[2], wom[3], wom[4], wom[5],
                   ln=sp["om"]["ln"], res=lom)
        lff = _mlp([[g[6], g[7]], [lff]], [None, wff[0][256:]], wff[1],
                   wff[2], wff[3], wff[4], wff[5],
                   ln=sp["ff"]["ln"], res=lff)
        pmm, pom, pff, pmo = _sc_scatter_all(
            zeros256, [lmm, lom, lff, lmo], sidx, [NMP, NMP, NMP, NOP])
        wn = _mlp_params(sp["mesh_node"])
        m = _mlp([[m], [pmm[0], pmm[1]], [pom[0], pom[1]], [pff[0], pff[1]]],
                 [wn[0][:128], wn[0][128:256], wn[0][256:384], wn[0][384:]],
                 wn[1], wn[2], wn[3], wn[4], wn[5],
                 ln=sp["mesh_node"]["ln"], res=m)
        wo = _mlp_params(sp["obj_node"])
        o = _mlp([[o], [pmo[0], pmo[1]]], [wo[0][:128], wo[0][128:]],
                 wo[1], wo[2], wo[3], wo[4], wo[5],
                 ln=sp["obj_node"]["ln"], res=o)

    w1, b1, w2, b2, w3, b3 = _mlp_params(pm["dec_mesh"])
    md = _mlp([[m]], [w1], b1, w2, b2,
              jnp.pad(w3, ((0, 0), (0, 125))), jnp.pad(b3, (0, 125)))
    w1, b1, w2, b2, w3, b3 = _mlp_params(pm["dec_obj"])
    od = _mlp([[o]], [w1], b1, w2, b2,
              jnp.pad(w3, ((0, 0), (0, 125))), jnp.pad(b3, (0, 125)))
    return md[:N_MESH, :3], od[:N_OBJ, :3]


# speedup vs baseline: 3.0627x; 1.0502x over previous
import functools

import jax
import jax.numpy as jnp
from jax import lax
from jax.experimental import pallas as pl
from jax.experimental.pallas import tpu as pltpu
from jax.experimental.pallas import tpu_sc as plsc

F32 = jnp.float32
LAT = 128
N_MESH, N_OBJ = 10000, 2000
NMP, NOP = 10240, 2048
E_MM_P, E_MO_P, E_FF_P = 163840, 16384, 8192
_NC, _NS, _NW = 2, 16, 32


def _dot(a, b):
    return lax.dot_general(a, b, (((1,), (0,)), ((), ())),
                           preferred_element_type=F32,
                           precision=lax.Precision.DEFAULT)


def _mlp(groups, w1s, b1, w2, b2, w3, b3, ln=None, res=None, block=2048):
    xs = [x for g in groups for x in g]
    R = xs[0].shape[0]
    sizes = [len(g) for g in groups]
    has_w = [w is not None for w in w1s]
    ws = [w for w in w1s if w is not None]
    nx, nw = len(xs), len(ws)

    def body(*refs):
        xr = refs[:nx]
        wr = refs[nx:nx + nw]
        b1r, w2r, b2r, w3r, b3r = refs[nx + nw:nx + nw + 5]
        p = nx + nw + 5
        if ln is not None:
            lgr, lbr = refs[p], refs[p + 1]
            p += 2
        if res is not None:
            rr = refs[p]
            p += 1
        out = refs[-1]
        h = None
        k = wi = 0
        for gi, sz in enumerate(sizes):
            acc = xr[k][...]
            for j in range(1, sz):
                acc = acc + xr[k + j][...]
            k += sz
            if has_w[gi]:
                acc = _dot(acc, wr[wi][...])
                wi += 1
            h = acc if h is None else h + acc
        h = jnp.maximum(h + b1r[...], 0.0)
        h = jnp.maximum(_dot(h, w2r[...]) + b2r[...], 0.0)
        y = _dot(h, w3r[...]) + b3r[...]
        if ln is not None:
            mu = jnp.mean(y, axis=-1, keepdims=True)
            var = jnp.mean((y - mu) ** 2, axis=-1, keepdims=True)
            y = (y - mu) * lax.rsqrt(var + 1e-5) * lgr[...] + lbr[...]
        if res is not None:
            y = y + rr[...]
        out[...] = y

    args = list(xs) + list(ws) + [b1.reshape(1, -1), w2, b2.reshape(1, -1),
                                  w3, b3.reshape(1, -1)]
    if ln is not None:
        args += [ln[0].reshape(1, -1), ln[1].reshape(1, -1)]
    if res is not None:
        args.append(res)
    in_specs = [pl.BlockSpec((block, x.shape[1]), lambda i: (i, 0)) for x in xs]
    in_specs += [pl.BlockSpec(w.shape, lambda i: (0, 0)) for w in ws]
    in_specs += [pl.BlockSpec(a.shape, lambda i: (0, 0)) for a in args[nx + nw:nx + nw + 5]]
    if ln is not None:
        in_specs += [pl.BlockSpec((1, LAT), lambda i: (0, 0))] * 2
    if res is not None:
        in_specs.append(pl.BlockSpec((block, LAT), lambda i: (i, 0)))
    return pl.pallas_call(
        body,
        grid=(R // block,),
        in_specs=in_specs,
        out_specs=pl.BlockSpec((block, LAT), lambda i: (i, 0)),
        out_shape=jax.ShapeDtypeStruct((R, LAT), F32),
    )(*args)


def _sc_gather_all(tables, tmap, idxs2d):
    nj = len(idxs2d)
    epads = [i.shape[0] * 128 for i in idxs2d]
    rows_w = [e // 128 // _NW for e in epads]
    joff = [0]
    for r in rows_w:
        joff.append(joff[-1] + r)
    mesh = plsc.VectorSubcoreMesh(core_axis_name="c", subcore_axis_name="s")

    nt = len(tables)
    nts = [t.shape[0] for t in tables]

    @functools.partial(
        pl.kernel, mesh=mesh,
        out_type=[jax.ShapeDtypeStruct((e, LAT), F32) for e in epads],
        scratch_types=[
            pltpu.VMEM_SHARED((NMP, LAT), F32),
            pltpu.VMEM((joff[-1], 128), jnp.int32),
            pltpu.VMEM((256, LAT), F32),
            pltpu.SemaphoreType.DMA,
            pltpu.SemaphoreType.DMA((2,)),
        ],
    )
    def k(*refs):
        trefs = refs[:nt]
        irefs = refs[nt:nt + nj]
        orefs = refs[nt + nj:nt + 2 * nj]
        sh_t, idx_all, rows_v, gsem, wsem = refs[nt + 2 * nj:]
        s = lax.axis_index("s")
        wid = s * _NC + lax.axis_index("c")
        icps = [pltpu.async_copy(irefs[j].at[pl.ds(wid * rows_w[j], rows_w[j])],
                                 idx_all.at[pl.ds(joff[j], rows_w[j])], gsem)
                for j in range(nj)]
        for cp in icps:
            cp.wait()
        for t in range(nt):
            rpt_t = nts[t] // _NS
            pltpu.sync_copy(trefs[t].at[pl.ds(s * rpt_t, rpt_t)],
                            sh_t.at[pl.ds(s * rpt_t, rpt_t)])
            plsc.subcore_barrier()
            for j in range(nj):
                if tmap[j] != t:
                    continue
                rows = rows_w[j]
                base = wid * rows

                def gbody(sg, _, j=j, base=base):
                    h = sg % 2

                    @pl.when(sg >= 2)
                    def _():
                        pltpu.make_async_copy(
                            rows_v.at[pl.ds(h * 128, 128)],
                            orefs[j].at[pl.ds((base + sg - 2) * 128, 128)],
                            wsem.at[h]).wait()

                    pltpu.async_copy(sh_t.at[idx_all.at[joff[j] + sg]],
                                     rows_v.at[pl.ds(h * 128, 128)],
                                     gsem).wait()
                    pltpu.async_copy(rows_v.at[pl.ds(h * 128, 128)],
                                     orefs[j].at[pl.ds((base + sg) * 128, 128)],
                                     wsem.at[h])
                    return 0

                lax.fori_loop(0, rows_w[j], gbody, 0)
                for q in range(max(rows_w[j] - 2, 0), rows_w[j]):
                    pltpu.make_async_copy(
                        rows_v.at[pl.ds((q % 2) * 128, 128)],
                        orefs[j].at[pl.ds((base + q) * 128, 128)],
                        wsem.at[q % 2]).wait()
            plsc.subcore_barrier()

    return k(*tables, *idxs2d)


def _sc_scatter_all(zeros, vals, idxs2d, out_rows):
    nj = len(vals)
    epads = [i.shape[0] * 128 for i in idxs2d]
    mesh = plsc.VectorSubcoreMesh(core_axis_name="c", subcore_axis_name="s")

    rows_w = [e // 128 // _NW for e in epads]
    joff = [0]
    for r in rows_w:
        joff.append(joff[-1] + r)

    @functools.partial(
        pl.kernel, mesh=mesh,
        out_type=[jax.ShapeDtypeStruct((2, r, LAT), F32) for r in out_rows],
        scratch_types=[
            pltpu.VMEM_SHARED((NMP, LAT), F32),
            pltpu.VMEM((joff[-1], 128), jnp.int32),
            pltpu.VMEM((256, LAT), F32),
            pltpu.SemaphoreType.DMA,
            pltpu.SemaphoreType.DMA((2,)),
        ],
    )
    def k(*refs):
        zr = refs[0]
        vrefs = refs[1:1 + nj]
        irefs = refs[1 + nj:1 + 2 * nj]
        orefs = refs[1 + 2 * nj:1 + 3 * nj]
        acc, idx_all, buf, sem, lsem = refs[1 + 3 * nj:]
        c = lax.axis_index("c")
        s = lax.axis_index("s")
        wid = s * _NC + c
        icps = [pltpu.async_copy(irefs[j].at[pl.ds(wid * rows_w[j], rows_w[j])],
                                 idx_all.at[pl.ds(joff[j], rows_w[j])], sem)
                for j in range(nj)]
        for cp in icps:
            cp.wait()
        for j in range(nj):
            rpt = out_rows[j] // _NS
            pltpu.sync_copy(zr, buf)
            off = 0
            while off < rpt:
                step = 256 if rpt - off >= 256 else 128
                pltpu.sync_copy(buf.at[pl.ds(0, step)],
                                acc.at[pl.ds(s * rpt + off, step)])
                off += step
            plsc.subcore_barrier()
            rows = rows_w[j]
            base = wid * rows
            pltpu.async_copy(vrefs[j].at[pl.ds(base * 128, 128)],
                             buf.at[pl.ds(0, 128)], lsem.at[0])

            def sbody(g, _, j=j, base=base, rows=rows):
                h = g % 2
                pltpu.make_async_copy(
                    vrefs[j].at[pl.ds((base + g) * 128, 128)],
                    buf.at[pl.ds(h * 128, 128)], lsem.at[h]).wait()

                @pl.when(g + 1 < rows)
                def _():
                    pltpu.async_copy(
                        vrefs[j].at[pl.ds((base + g + 1) * 128, 128)],
                        buf.at[pl.ds((1 - h) * 128, 128)], lsem.at[1 - h])

                pltpu.sync_copy(buf.at[pl.ds(h * 128, 128)],
                                acc.at[idx_all.at[joff[j] + g]], add=True)
                return 0

            lax.fori_loop(0, rows, sbody, 0)
            plsc.subcore_barrier()
            pltpu.sync_copy(acc.at[pl.ds(s * rpt, rpt)],
                            orefs[j].at[c, pl.ds(s * rpt, rpt)])
            plsc.subcore_barrier()

    return k(zeros, *vals, *idxs2d)


def _pad_rows(x, rows, fill=0.0):
    return jnp.pad(x, ((0, rows - x.shape[0]), (0, 0)), constant_values=fill)


def _pad_idx(idx, n, fill):
    return jnp.pad(idx, (0, n - idx.shape[0]), constant_values=fill)


def _mlp_params(p):
    (w1, b1), (w2, b2), (w3, b3) = p["layers"]
    return w1, b1, w2, b2, w3, b3


def kernel(mesh_features, obj_features, mesh_kinematic, obj_kinematic,
           index_mm, index_mo, index_om, index_ff, e_mm, e_mo, e_ff, params):
    pm = params
    m_kin = jax.nn.one_hot(mesh_kinematic, 3, dtype=F32)
    o_kin = jax.nn.one_hot(obj_kinematic, 3, dtype=F32)
    m_in = jnp.concatenate([mesh_features, m_kin], axis=-1)
    o_in = jnp.concatenate([obj_features, o_kin], axis=-1)
    m_in = (m_in - pm["node_mean"]) / pm["node_std"]
    o_in = (o_in - pm["node_mean"]) / pm["node_std"]
    nk = jax.random.key(42)
    m_in = m_in.at[:, :3].add(
        1e-05 * jax.random.normal(jax.random.fold_in(nk, 0), (N_MESH, 3), F32))
    o_in = o_in.at[:, :3].add(
        1e-05 * jax.random.normal(jax.random.fold_in(nk, 1), (N_OBJ, 3), F32))
    m_in = _pad_rows(jnp.pad(m_in, ((0, 0), (0, 5))), NMP)
    o_in = _pad_rows(jnp.pad(o_in, ((0, 0), (0, 5))), NOP)

    e_mm_p = _pad_rows(e_mm, E_MM_P)
    e_mo_p = _pad_rows(e_mo, E_MO_P)
    e_ff_p = _pad_rows(jnp.pad(e_ff, ((0, 0), (0, 6))), E_FF_P)

    imm0 = _pad_idx(index_mm[0], E_MM_P, 0)
    imm1 = _pad_idx(index_mm[1], E_MM_P, N_MESH)
    imo0 = _pad_idx(index_mo[0], E_MO_P, 0)
    imo1 = _pad_idx(index_mo[1], E_MO_P, N_OBJ)
    iom0 = _pad_idx(index_om[0], E_MO_P, 0)
    iom1 = _pad_idx(index_om[1], E_MO_P, N_MESH)
    iff0 = _pad_idx(index_ff[0], E_FF_P, 0)
    iff1 = _pad_idx(index_ff[1], E_FF_P, N_MESH)
    i2 = lambda ix: ix.reshape(-1, 128)
    gidx = [i2(imm0), i2(imm1), i2(imo0), i2(imo1),
            i2(iom0), i2(iom1), i2(iff0), i2(iff1)]
    sidx = [i2(imm1), i2(iom1), i2(iff1), i2(imo1)]
    zeros256 = jnp.zeros((256, LAT), F32)

    def fold(enc, mean, std, pad_to):
        w1, b1, w2, b2, w3, b3 = _mlp_params(enc)
        w1f = w1 / std[:, None]
        b1f = b1 - (mean / std) @ w1
        w1f = jnp.pad(w1f, ((0, pad_to - w1f.shape[0]), (0, 0)))
        return w1f, b1f, w2, b2, w3, b3

    w1, b1, w2, b2, w3, b3 = _mlp_params(pm["enc_mesh"])
    m = _mlp([[m_in]], [jnp.pad(w1, ((0, 5), (0, 0)))], b1, w2, b2, w3, b3,
             ln=pm["enc_mesh"]["ln"])
    w1, b1, w2, b2, w3, b3 = _mlp_params(pm["enc_obj"])
    o = _mlp([[o_in]], [jnp.pad(w1, ((0, 5), (0, 0)))], b1, w2, b2, w3, b3,
             ln=pm["enc_obj"]["ln"])
    w1f, b1f, w2, b2, w3, b3 = fold(pm["enc_mm"], pm["edge_mean"], pm["edge_std"], 8)
    lmm = _mlp([[e_mm_p]], [w1f], b1f, w2, b2, w3, b3, ln=pm["enc_mm"]["ln"])
    w1f, b1f, w2, b2, w3, b3 = fold(pm["enc_mo"], pm["edge_mean"], pm["edge_std"], 8)
    lmo = _mlp([[e_mo_p]], [w1f], b1f, w2, b2, w3, b3, ln=pm["enc_mo"]["ln"])
    w1f, b1f, w2, b2, w3, b3 = fold(pm["enc_om"], pm["edge_mean"], pm["edge_std"], 8)
    lom = _mlp([[e_mo_p]], [w1f], b1f, w2, b2, w3, b3, ln=pm["enc_om"]["ln"])
    w1f, b1f, w2, b2, w3, b3 = fold(pm["enc_ff"], pm["face_mean"], pm["face_std"], 40)
    lff = _mlp([[e_ff_p]], [w1f], b1f, w2, b2, w3, b3, ln=pm["enc_ff"]["ln"])

    for sp in pm["steps"]:
        wmm = _mlp_params(sp["mm"])
        wmo = _mlp_params(sp["mo"])
        wom = _mlp_params(sp["om"])
        wff = _mlp_params(sp["ff"])
        g = _sc_gather_all([m, o], [0, 0, 0, 1, 1, 0, 0, 0], gidx)
        lmm = _mlp([[g[0]], [g[1]], [lmm]],
                   [wmm[0][:128], wmm[0][128:256], wmm[0][256:]], wmm[1],
                   wmm[2], wmm[3], wmm[4], wmm[5],
                   ln=sp["mm"]["ln"], res=lmm)
        lmo = _mlp([[g[2]], [g[3]], [lmo]],
                   [wmo[0][:128], wmo[0][128:256], wmo[0][256:]], wmo[1],
                   wmo[2], wmo[3], wmo[4], wmo[5],
                   ln=sp["mo"]["ln"], res=lmo)
        lom = _mlp([[g[4]], [g[5]], [lom]],
                   [wom[0][:128], wom[0][128:256], wom[0][256:]], wom[1],
                   wom[2], wom[3], wom[4], wom[5],
                   ln=sp["om"]["ln"], res=lom)
        lff = _mlp([[g[6]], [g[7]], [lff]],
                   [wff[0][:128], wff[0][128:256], wff[0][256:]], wff[1],
                   wff[2], wff[3], wff[4], wff[5],
                   ln=sp["ff"]["ln"], res=lff)
        pmm, pom, pff, pmo = _sc_scatter_all(
            zeros256, [lmm, lom, lff, lmo], sidx, [NMP, NMP, NMP, NOP])
        wn = _mlp_params(sp["mesh_node"])
        m = _mlp([[m], [pmm[0], pmm[1]], [pom[0], pom[1]], [pff[0], pff[1]]],
                 [wn[0][:128], wn[0][128:256], wn[0][256:384], wn[0][384:]],
                 wn[1], wn[2], wn[3], wn[4], wn[5],
                 ln=sp["mesh_node"]["ln"], res=m)
        wo = _mlp_params(sp["obj_node"])
        o = _mlp([[o], [pmo[0], pmo[1]]], [wo[0][:128], wo[0][128:]],
                 wo[1], wo[2], wo[3], wo[4], wo[5],
                 ln=sp["obj_node"]["ln"], res=o)

    w1, b1, w2, b2, w3, b3 = _mlp_params(pm["dec_mesh"])
    md = _mlp([[m]], [w1], b1, w2, b2,
              jnp.pad(w3, ((0, 0), (0, 125))), jnp.pad(b3, (0, 125)))
    w1, b1, w2, b2, w3, b3 = _mlp_params(pm["dec_obj"])
    od = _mlp([[o]], [w1], b1, w2, b2,
              jnp.pad(w3, ((0, 0), (0, 125))), jnp.pad(b3, (0, 125)))
    return md[:N_MESH, :3], od[:N_OBJ, :3]
